# Initial kernel scaffold; baseline (speedup 1.0000x reference)
import functools

import jax
import jax.numpy as jnp
from jax import lax
from jax.experimental import pallas as pl
from jax.experimental.pallas import tpu as pltpu
from jax.experimental.pallas import tpu_sc as plsc

N = 41472
E = 829440
H = 96
EMB = 16
DE = 16

_GW = 32
_GCHUNK = (2 * E) // _GW
_GSUB = 6480
_GNSUB = _GCHUNK // _GSUB


def _gather_cls(nodes, eflat):
    mesh = plsc.VectorSubcoreMesh(core_axis_name="c", subcore_axis_name="s")

    @functools.partial(
        pl.kernel,
        out_type=jax.ShapeDtypeStruct((2 * E,), jnp.int32),
        mesh=mesh,
        scratch_types=[
            pltpu.VMEM((N,), jnp.int32),
            pltpu.VMEM((_GSUB,), jnp.int32),
            pltpu.VMEM((_GSUB,), jnp.int32),
        ],
    )
    def k(nodes_hbm, eflat_hbm, out_hbm, tbl, ibuf, obuf):
        cid = lax.axis_index("c")
        sid = lax.axis_index("s")
        wid = sid * 2 + cid
        base = wid * _GCHUNK
        pltpu.sync_copy(nodes_hbm, tbl)

        for sc in range(_GNSUB):
            off = base + sc * _GSUB
            pltpu.sync_copy(eflat_hbm.at[pl.ds(off, _GSUB)], ibuf)

            def body(i, carry):
                idx = ibuf[pl.ds(i * 16, 16)]
                g = plsc.load_gather(tbl, [idx])
                obuf[pl.ds(i * 16, 16)] = g
                return carry

            lax.fori_loop(0, _GSUB // 16, body, 0)
            pltpu.sync_copy(obuf, out_hbm.at[pl.ds(off, _GSUB)])

    return k(nodes, eflat)


_SROWS_PER_TILE = (E // 128) // 16
_SCHUNK_ROWS = 9
_SCHUNK = _SCHUNK_ROWS * 128
_SNCHUNK = _SROWS_PER_TILE // _SCHUNK_ROWS
_SZROWS = 162
_NPT = N // 16


def _scatter_agg(dst2d, msgs2):
    mesh = plsc.VectorSubcoreMesh(core_axis_name="c", subcore_axis_name="s")

    @functools.partial(
        pl.kernel,
        out_type=jax.ShapeDtypeStruct((2, N, 48), jnp.float32),
        mesh=mesh,
        scratch_types=[
            pltpu.VMEM_SHARED((N, 48), jnp.float32),
            pltpu.VMEM((_SZROWS, 48), jnp.float32),
            pltpu.VMEM((_SCHUNK, 48), jnp.float32),
            pltpu.VMEM((_SCHUNK_ROWS, 128), jnp.int32),
        ],
    )
    def k(dst2d_hbm, msgs2_hbm, out_hbm, acc, zbuf, dbuf, ibuf):
        cid = lax.axis_index("c")
        sid = lax.axis_index("s")

        zeros16 = jnp.zeros((16,), jnp.float32)

        def zrow(i, carry):
            zbuf[i, pl.ds(0, 16)] = zeros16
            zbuf[i, pl.ds(16, 16)] = zeros16
            zbuf[i, pl.ds(32, 16)] = zeros16
            return carry

        lax.fori_loop(0, _SZROWS, zrow, 0)
        for t in range(_NPT // _SZROWS):
            pltpu.sync_copy(zbuf, acc.at[pl.ds(sid * _NPT + t * _SZROWS, _SZROWS)])
        plsc.subcore_barrier()

        def chunk(t, carry):
            row0 = sid * _SROWS_PER_TILE + t * _SCHUNK_ROWS
            ebase = row0 * 128
            pltpu.sync_copy(msgs2_hbm.at[cid, pl.ds(ebase, _SCHUNK)], dbuf)
            pltpu.sync_copy(dst2d_hbm.at[pl.ds(row0, _SCHUNK_ROWS)], ibuf)
            for j in range(_SCHUNK_ROWS):
                pltpu.sync_copy(dbuf.at[pl.ds(j * 128, 128)],
                                acc.at[ibuf.at[j]], add=True)
            return carry

        lax.fori_loop(0, _SNCHUNK, chunk, 0)
        plsc.subcore_barrier()

        pltpu.sync_copy(acc.at[pl.ds(sid * _NPT, _NPT)],
                        out_hbm.at[cid, pl.ds(sid * _NPT, _NPT)])

    return k(dst2d, msgs2)


_EB = 2048
_ENB = E // _EB


def _edge_body(cs_ref, cd_ref, ef_ref, emb_ref, mW0_ref, b0_ref, mW1_ref,
               b1_ref, mW2a_ref, mW2b_ref, b2a_ref, b2b_ref, out_ref):
    f32 = jnp.float32
    dims11 = (((1,), (1,)), ((), ()))
    dims10 = (((1,), (0,)), ((), ()))
    cs = cs_ref[0, 0, :]
    cd = cd_ref[0, 0, :]
    iota10 = lax.broadcasted_iota(jnp.int32, (1, 10), 1)
    oh_s = (cs[:, None] == iota10).astype(f32)
    oh_d = (cd[:, None] == iota10).astype(f32)
    emb = emb_ref[...]
    TA = lax.dot_general(emb, mW0_ref[:, 0:EMB], dims11,
                         preferred_element_type=f32)
    TB = lax.dot_general(emb, mW0_ref[:, EMB:2 * EMB], dims11,
                         preferred_element_type=f32)
    h0 = (lax.dot_general(oh_s, TA, dims10, preferred_element_type=f32)
          + lax.dot_general(oh_d, TB, dims10, preferred_element_type=f32)
          + lax.dot_general(ef_ref[...], mW0_ref[:, 2 * EMB:], dims11,
                            preferred_element_type=f32)
          + b0_ref[...])
    h1 = jnp.maximum(h0, 0.0)
    h2 = jnp.maximum(
        lax.dot_general(h1, mW1_ref[...], dims11, preferred_element_type=f32)
        + b1_ref[...], 0.0)
    out_ref[0] = lax.dot_general(h2, mW2a_ref[...], dims11,
                                 preferred_element_type=f32) + b2a_ref[...]
    out_ref[1] = lax.dot_general(h2, mW2b_ref[...], dims11,
                                 preferred_element_type=f32) + b2b_ref[...]


def _edge_mlp(cs3, cd3, ef, emb_s, mW0, b0r, mW1, b1r, mW2a, mW2b, b2ar, b2br):
    full = lambda shape: pl.BlockSpec(shape, lambda i, _s=shape: tuple(0 for _ in _s))
    return pl.pallas_call(
        _edge_body,
        grid=(_ENB,),
        in_specs=[
            pl.BlockSpec((1, 1, _EB), lambda i: (i, 0, 0)),
            pl.BlockSpec((1, 1, _EB), lambda i: (i, 0, 0)),
            pl.BlockSpec((_EB, DE), lambda i: (i, 0)),
            full((10, EMB)),
            full((H, 2 * EMB + DE)),
            full((1, H)),
            full((H, H)),
            full((1, H)),
            full((48, H)),
            full((48, H)),
            full((1, 48)),
            full((1, 48)),
        ],
        out_specs=pl.BlockSpec((2, _EB, 48), lambda i: (0, i, 0)),
        out_shape=jax.ShapeDtypeStruct((2, E, 48), jnp.float32),
        compiler_params=pltpu.CompilerParams(
            dimension_semantics=("arbitrary",)),
    )(cs3, cd3, ef, emb_s, mW0, b0r, mW1, b1r, mW2a, mW2b, b2ar, b2br)


_RB = 1296
_RNB = N // _RB


def _post_body(agg_ref, pz_ref, sh_ref, sc_ref, pW0_ref, pb0_ref, pW1_ref,
               pb1_ref, pW2_ref, pb2_ref, Wi_ref, Wf_ref, Wg_ref, Wo_ref,
               Ui_ref, Uf_ref, Ug_ref, Uo_ref, bi_ref, bf_ref, bg_ref,
               bo_ref, oW_ref, ob_ref, h_ref, c_ref, o_ref):
    f32 = jnp.float32
    dims11 = (((1,), (1,)), ((), ()))

    def dot(x, w):
        return lax.dot_general(x, w, dims11, preferred_element_type=f32)

    a0 = agg_ref[0]
    a1 = agg_ref[1]
    g0 = (dot(a0, pW0_ref[:, 0:48]) + dot(a1, pW0_ref[:, 48:96])
          + dot(pz_ref[...], pW0_ref[:, 96:112]) + pb0_ref[...])
    h = jnp.maximum(g0, 0.0)
    h = jnp.maximum(dot(h, pW1_ref[...]) + pb1_ref[...], 0.0)
    hp = dot(h, pW2_ref[...]) + pb2_ref[...]
    sh = sh_ref[...]
    ii = jax.nn.sigmoid(dot(hp, Wi_ref[...]) + dot(sh, Ui_ref[...]) + bi_ref[...])
    ff = jax.nn.sigmoid(dot(hp, Wf_ref[...]) + dot(sh, Uf_ref[...]) + bf_ref[...])
    gg = jnp.tanh(dot(hp, Wg_ref[...]) + dot(sh, Ug_ref[...]) + bg_ref[...])
    oo = jax.nn.sigmoid(dot(hp, Wo_ref[...]) + dot(sh, Uo_ref[...]) + bo_ref[...])
    cn = ff * sc_ref[...] + ii * gg
    hn = oo * jnp.tanh(cn)
    h_ref[...] = hn
    c_ref[...] = cn
    o_ref[...] = dot(hn, oW_ref[...]) + ob_ref[...]


def _post(agg, puzzle, sh, sc, pW0, pb0r, pW1, pb1r, pW2, pb2r, Ws, Us, bs,
          oW, obr):
    full = lambda shape: pl.BlockSpec(shape, lambda i, _s=shape: tuple(0 for _ in _s))
    return pl.pallas_call(
        _post_body,
        grid=(_RNB,),
        in_specs=[
            pl.BlockSpec((2, _RB, 48), lambda i: (0, i, 0)),
            pl.BlockSpec((_RB, EMB), lambda i: (i, 0)),
            pl.BlockSpec((_RB, H), lambda i: (i, 0)),
            pl.BlockSpec((_RB, H), lambda i: (i, 0)),
            full((H, H + EMB)),
            full((1, H)),
            full((H, H)),
            full((1, H)),
            full((H, H)),
            full((1, H)),
            *[full((H, H)) for _ in range(8)],
            *[full((1, H)) for _ in range(4)],
            full((10, H)),
            full((1, 10)),
        ],
        out_specs=[
            pl.BlockSpec((_RB, H), lambda i: (i, 0)),
            pl.BlockSpec((_RB, H), lambda i: (i, 0)),
            pl.BlockSpec((_RB, 10), lambda i: (i, 0)),
        ],
        out_shape=[
            jax.ShapeDtypeStruct((N, H), jnp.float32),
            jax.ShapeDtypeStruct((N, H), jnp.float32),
            jax.ShapeDtypeStruct((N, 10), jnp.float32),
        ],
        compiler_params=pltpu.CompilerParams(
            dimension_semantics=("arbitrary",)),
    )(agg, puzzle, sh, sc, pW0, pb0r, pW1, pb1r, pW2, pb2r, *Ws, *Us, *bs,
      oW, obr)


def kernel(puzzle, nodes, edges, edge_features, state_h, state_c, first, emb,
           mW0, mb0, mW1, mb1, mW2, mb2, pW0, pb0, pW1, pb1, pW2, pb2, W_ih,
           W_hh, b_ih, b_hh, oW, ob):
    f32 = jnp.float32
    nodes = nodes.astype(jnp.int32)
    eflat = edges.astype(jnp.int32).reshape(2 * E)
    dst2d = edges[1].astype(jnp.int32).reshape(E // 128, 128)

    cls = _gather_cls(nodes, eflat)
    cs3 = cls[:E].reshape(_ENB, 1, _EB)
    cd3 = cls[E:].reshape(_ENB, 1, _EB)

    emb_s = emb * jnp.asarray(first, f32)
    msgs2 = _edge_mlp(
        cs3, cd3, edge_features, emb_s, mW0, mb0.reshape(1, H), mW1,
        mb1.reshape(1, H), mW2[:48], mW2[48:], mb2[:48].reshape(1, 48),
        mb2[48:].reshape(1, 48))

    agg = _scatter_agg(dst2d, msgs2)

    b = (b_ih + b_hh).reshape(1, 4 * H)
    Ws = [W_ih[i * H:(i + 1) * H] for i in range(4)]
    Us = [W_hh[i * H:(i + 1) * H] for i in range(4)]
    bs = [b[:, i * H:(i + 1) * H] for i in range(4)]
    h_new, c_new, out10 = _post(
        agg, puzzle, state_h, state_c, pW0, pb0.reshape(1, H), pW1,
        pb1.reshape(1, H), pW2, pb2.reshape(1, H), Ws, Us, bs, oW,
        ob.reshape(1, 10))
    return (h_new, c_new, out10.reshape(-1, 81, 10))


# speedup vs baseline: 2.9122x; 2.9122x over previous
import functools

import jax
import jax.numpy as jnp
from jax import lax
from jax.experimental import pallas as pl
from jax.experimental.pallas import tpu as pltpu
from jax.experimental.pallas import tpu_sc as plsc

N = 41472
E = 829440
H = 96
EMB = 16
DE = 16

_GW = 32
_GCHUNK = (2 * E) // _GW
_GSUB = 6480
_GNSUB = _GCHUNK // _GSUB


def _gather_cls(nodes, eflat):
    mesh = plsc.VectorSubcoreMesh(core_axis_name="c", subcore_axis_name="s")

    @functools.partial(
        pl.kernel,
        out_type=jax.ShapeDtypeStruct((2 * E,), jnp.int32),
        mesh=mesh,
        scratch_types=[
            pltpu.VMEM((_GSUB,), jnp.int32),
            pltpu.VMEM((_GSUB,), jnp.int32),
            pltpu.SemaphoreType.DMA,
        ],
    )
    def k(nodes_hbm, eflat_hbm, out_hbm, ibuf, obuf, sem):
        cid = lax.axis_index("c")
        sid = lax.axis_index("s")
        wid = sid * 2 + cid
        base = wid * _GCHUNK

        for sc in range(_GNSUB):
            off = base + sc * _GSUB
            pltpu.sync_copy(eflat_hbm.at[pl.ds(off, _GSUB)], ibuf)
            pltpu.async_copy(nodes_hbm.at[ibuf], obuf, sem).wait()
            pltpu.sync_copy(obuf, out_hbm.at[pl.ds(off, _GSUB)])

    return k(nodes, eflat)


_SG = 24
_SROWS_PER_TILE = (E // 128) // 16
_SCHUNK_ROWS = 9
_SCHUNK = _SCHUNK_ROWS * 128
_SNCHUNK = _SROWS_PER_TILE // _SCHUNK_ROWS
_SZROWS = 162
_NPT = N // 16


def _scatter_agg(dst2d, msgs2):
    mesh = plsc.VectorSubcoreMesh(core_axis_name="c", subcore_axis_name="s")

    @functools.partial(
        pl.kernel,
        out_type=jax.ShapeDtypeStruct((4, N, _SG), jnp.float32),
        mesh=mesh,
        scratch_types=[
            pltpu.VMEM_SHARED((N, _SG), jnp.float32),
            pltpu.VMEM((_SZROWS, _SG), jnp.float32),
            pltpu.VMEM((_SCHUNK, _SG), jnp.float32),
            pltpu.VMEM((_SCHUNK_ROWS, 128), jnp.int32),
        ],
        compiler_params=pltpu.CompilerParams(use_tc_tiling_on_sc=False),
    )
    def k(dst2d_hbm, msgs2_hbm, out_hbm, acc, zbuf, dbuf, ibuf):
        cid = lax.axis_index("c")
        sid = lax.axis_index("s")

        zeros16 = jnp.zeros((16,), jnp.float32)

        def zrow(i, carry):
            zbuf[i, pl.ds(0, 16)] = zeros16
            zbuf[i, pl.ds(8, 16)] = zeros16
            return carry

        lax.fori_loop(0, _SZROWS, zrow, 0)

        for p in range(2):
            grp = cid * 2 + p
            for t in range(_NPT // _SZROWS):
                pltpu.sync_copy(
                    zbuf, acc.at[pl.ds(sid * _NPT + t * _SZROWS, _SZROWS)])
            plsc.subcore_barrier()

            def chunk(t, carry):
                row0 = sid * _SROWS_PER_TILE + t * _SCHUNK_ROWS
                ebase = row0 * 128
                pltpu.sync_copy(msgs2_hbm.at[grp, pl.ds(ebase, _SCHUNK)], dbuf)
                pltpu.sync_copy(dst2d_hbm.at[pl.ds(row0, _SCHUNK_ROWS)], ibuf)
                for j in range(_SCHUNK_ROWS):
                    pltpu.sync_copy(dbuf.at[pl.ds(j * 128, 128)],
                                    acc.at[ibuf.at[j]], add=True)
                return carry

            lax.fori_loop(0, _SNCHUNK, chunk, 0)
            plsc.subcore_barrier()

            pltpu.sync_copy(acc.at[pl.ds(sid * _NPT, _NPT)],
                            out_hbm.at[grp, pl.ds(sid * _NPT, _NPT)])

    return k(dst2d, msgs2)


_EB = 2048
_ENB = E // _EB


def _edge_body(cs_ref, cd_ref, ef_ref, emb_ref, mW0_ref, b0_ref, mW1_ref,
               b1_ref, mW2_ref, b2_ref, out_ref):
    f32 = jnp.float32
    dims11 = (((1,), (1,)), ((), ()))
    dims10 = (((1,), (0,)), ((), ()))
    cs = cs_ref[0, 0, :]
    cd = cd_ref[0, 0, :]
    iota10 = lax.broadcasted_iota(jnp.int32, (1, 10), 1)
    oh_s = (cs[:, None] == iota10).astype(f32)
    oh_d = (cd[:, None] == iota10).astype(f32)
    emb = emb_ref[...]
    TA = lax.dot_general(emb, mW0_ref[:, 0:EMB], dims11,
                         preferred_element_type=f32)
    TB = lax.dot_general(emb, mW0_ref[:, EMB:2 * EMB], dims11,
                         preferred_element_type=f32)
    h0 = (lax.dot_general(oh_s, TA, dims10, preferred_element_type=f32)
          + lax.dot_general(oh_d, TB, dims10, preferred_element_type=f32)
          + lax.dot_general(ef_ref[...], mW0_ref[:, 2 * EMB:], dims11,
                            preferred_element_type=f32)
          + b0_ref[...])
    h1 = jnp.maximum(h0, 0.0)
    h2 = jnp.maximum(
        lax.dot_general(h1, mW1_ref[...], dims11, preferred_element_type=f32)
        + b1_ref[...], 0.0)
    for g in range(4):
        out_ref[g] = lax.dot_general(
            h2, mW2_ref[pl.ds(g * _SG, _SG), :], dims11,
            preferred_element_type=f32) + b2_ref[:, pl.ds(g * _SG, _SG)]


def _edge_mlp(cs3, cd3, ef, emb_s, mW0, b0r, mW1, b1r, mW2, b2r):
    full = lambda shape: pl.BlockSpec(shape, lambda i, _s=shape: tuple(0 for _ in _s))
    return pl.pallas_call(
        _edge_body,
        grid=(_ENB,),
        in_specs=[
            pl.BlockSpec((1, 1, _EB), lambda i: (i, 0, 0)),
            pl.BlockSpec((1, 1, _EB), lambda i: (i, 0, 0)),
            pl.BlockSpec((_EB, DE), lambda i: (i, 0)),
            full((10, EMB)),
            full((H, 2 * EMB + DE)),
            full((1, H)),
            full((H, H)),
            full((1, H)),
            full((H, H)),
            full((1, H)),
        ],
        out_specs=pl.BlockSpec((4, _EB, _SG), lambda i: (0, i, 0)),
        out_shape=jax.ShapeDtypeStruct((4, E, _SG), jnp.float32),
        compiler_params=pltpu.CompilerParams(
            dimension_semantics=("arbitrary",)),
    )(cs3, cd3, ef, emb_s, mW0, b0r, mW1, b1r, mW2, b2r)


_RB = 1296
_RNB = N // _RB


def _post_body(agg_ref, pz_ref, sh_ref, sc_ref, pW0_ref, pb0_ref, pW1_ref,
               pb1_ref, pW2_ref, pb2_ref, Wi_ref, Wf_ref, Wg_ref, Wo_ref,
               Ui_ref, Uf_ref, Ug_ref, Uo_ref, bi_ref, bf_ref, bg_ref,
               bo_ref, oW_ref, ob_ref, h_ref, c_ref, o_ref):
    f32 = jnp.float32
    dims11 = (((1,), (1,)), ((), ()))

    def dot(x, w):
        return lax.dot_general(x, w, dims11, preferred_element_type=f32)

    g0 = (dot(agg_ref[0], pW0_ref[:, 0:_SG])
          + dot(agg_ref[1], pW0_ref[:, _SG:2 * _SG])
          + dot(agg_ref[2], pW0_ref[:, 2 * _SG:3 * _SG])
          + dot(agg_ref[3], pW0_ref[:, 3 * _SG:4 * _SG])
          + dot(pz_ref[...], pW0_ref[:, 96:112]) + pb0_ref[...])
    h = jnp.maximum(g0, 0.0)
    h = jnp.maximum(dot(h, pW1_ref[...]) + pb1_ref[...], 0.0)
    hp = dot(h, pW2_ref[...]) + pb2_ref[...]
    sh = sh_ref[...]
    ii = jax.nn.sigmoid(dot(hp, Wi_ref[...]) + dot(sh, Ui_ref[...]) + bi_ref[...])
    ff = jax.nn.sigmoid(dot(hp, Wf_ref[...]) + dot(sh, Uf_ref[...]) + bf_ref[...])
    gg = jnp.tanh(dot(hp, Wg_ref[...]) + dot(sh, Ug_ref[...]) + bg_ref[...])
    oo = jax.nn.sigmoid(dot(hp, Wo_ref[...]) + dot(sh, Uo_ref[...]) + bo_ref[...])
    cn = ff * sc_ref[...] + ii * gg
    hn = oo * jnp.tanh(cn)
    h_ref[...] = hn
    c_ref[...] = cn
    o_ref[...] = dot(hn, oW_ref[...]) + ob_ref[...]


def _post(agg, puzzle, sh, sc, pW0, pb0r, pW1, pb1r, pW2, pb2r, Ws, Us, bs,
          oW, obr):
    full = lambda shape: pl.BlockSpec(shape, lambda i, _s=shape: tuple(0 for _ in _s))
    return pl.pallas_call(
        _post_body,
        grid=(_RNB,),
        in_specs=[
            pl.BlockSpec((4, _RB, _SG), lambda i: (0, i, 0)),
            pl.BlockSpec((_RB, EMB), lambda i: (i, 0)),
            pl.BlockSpec((_RB, H), lambda i: (i, 0)),
            pl.BlockSpec((_RB, H), lambda i: (i, 0)),
            full((H, H + EMB)),
            full((1, H)),
            full((H, H)),
            full((1, H)),
            full((H, H)),
            full((1, H)),
            *[full((H, H)) for _ in range(8)],
            *[full((1, H)) for _ in range(4)],
            full((10, H)),
            full((1, 10)),
        ],
        out_specs=[
            pl.BlockSpec((_RB, H), lambda i: (i, 0)),
            pl.BlockSpec((_RB, H), lambda i: (i, 0)),
            pl.BlockSpec((_RB, 10), lambda i: (i, 0)),
        ],
        out_shape=[
            jax.ShapeDtypeStruct((N, H), jnp.float32),
            jax.ShapeDtypeStruct((N, H), jnp.float32),
            jax.ShapeDtypeStruct((N, 10), jnp.float32),
        ],
        compiler_params=pltpu.CompilerParams(
            dimension_semantics=("arbitrary",)),
    )(agg, puzzle, sh, sc, pW0, pb0r, pW1, pb1r, pW2, pb2r, *Ws, *Us, *bs,
      oW, obr)


def kernel(puzzle, nodes, edges, edge_features, state_h, state_c, first, emb,
           mW0, mb0, mW1, mb1, mW2, mb2, pW0, pb0, pW1, pb1, pW2, pb2, W_ih,
           W_hh, b_ih, b_hh, oW, ob):
    f32 = jnp.float32
    nodes = nodes.astype(jnp.int32)
    eflat = edges.astype(jnp.int32).reshape(2 * E)
    dst2d = edges[1].astype(jnp.int32).reshape(E // 128, 128)

    cls = _gather_cls(nodes, eflat)
    cs3 = cls[:E].reshape(_ENB, 1, _EB)
    cd3 = cls[E:].reshape(_ENB, 1, _EB)

    emb_s = emb * jnp.asarray(first, f32)
    msgs2 = _edge_mlp(
        cs3, cd3, edge_features, emb_s, mW0, mb0.reshape(1, H), mW1,
        mb1.reshape(1, H), mW2, mb2.reshape(1, H))

    agg = _scatter_agg(dst2d, msgs2)

    b = (b_ih + b_hh).reshape(1, 4 * H)
    Ws = [W_ih[i * H:(i + 1) * H] for i in range(4)]
    Us = [W_hh[i * H:(i + 1) * H] for i in range(4)]
    bs = [b[:, i * H:(i + 1) * H] for i in range(4)]
    h_new, c_new, out10 = _post(
        agg, puzzle, state_h, state_c, pW0, pb0.reshape(1, H), pW1,
        pb1.reshape(1, H), pW2, pb2.reshape(1, H), Ws, Us, bs, oW,
        ob.reshape(1, 10))
    return (h_new, c_new, out10.reshape(-1, 81, 10))


# speedup vs baseline: 5.9506x; 2.0433x over previous
import functools

import jax
import jax.numpy as jnp
from jax import lax
from jax.experimental import pallas as pl
from jax.experimental.pallas import tpu as pltpu
from jax.experimental.pallas import tpu_sc as plsc

N = 41472
E = 829440
H = 96
EMB = 16
DE = 16

_GW = 32
_GCHUNK = (2 * E) // _GW
_GSUB = 6480
_GNSUB = _GCHUNK // _GSUB


def _gather_cls(nodes, eflat):
    mesh = plsc.VectorSubcoreMesh(core_axis_name="c", subcore_axis_name="s")

    @functools.partial(
        pl.kernel,
        out_type=jax.ShapeDtypeStruct((2 * E,), jnp.int32),
        mesh=mesh,
        scratch_types=[
            pltpu.VMEM((_GSUB,), jnp.int32),
            pltpu.VMEM((_GSUB,), jnp.int32),
            pltpu.SemaphoreType.DMA,
        ],
    )
    def k(nodes_hbm, eflat_hbm, out_hbm, ibuf, obuf, sem):
        cid = lax.axis_index("c")
        sid = lax.axis_index("s")
        wid = sid * 2 + cid
        base = wid * _GCHUNK

        for sc in range(_GNSUB):
            off = base + sc * _GSUB
            pltpu.sync_copy(eflat_hbm.at[pl.ds(off, _GSUB)], ibuf)
            pltpu.async_copy(nodes_hbm.at[ibuf], obuf, sem).wait()
            pltpu.sync_copy(obuf, out_hbm.at[pl.ds(off, _GSUB)])

    return k(nodes, eflat)


_SG = 24
_SROWS_PER_TILE = (E // 128) // 16
_SCHUNK_ROWS = 9
_SCHUNK = _SCHUNK_ROWS * 128
_SNCHUNK = _SROWS_PER_TILE // _SCHUNK_ROWS
_SZROWS = 162
_NPT = N // 16


def _scatter_agg(dst2d, msgs2):
    mesh = plsc.VectorSubcoreMesh(core_axis_name="c", subcore_axis_name="s")

    @functools.partial(
        pl.kernel,
        out_type=jax.ShapeDtypeStruct((4, N, _SG), jnp.float32),
        mesh=mesh,
        scratch_types=[
            pltpu.VMEM_SHARED((N, _SG), jnp.float32),
            pltpu.VMEM((_SZROWS, _SG), jnp.float32),
            pltpu.VMEM((_SCHUNK, _SG), jnp.float32),
            pltpu.VMEM((_SCHUNK_ROWS, 128), jnp.int32),
        ],
        compiler_params=pltpu.CompilerParams(use_tc_tiling_on_sc=False),
    )
    def k(dst2d_hbm, msgs2_hbm, out_hbm, acc, zbuf, dbuf, ibuf):
        cid = lax.axis_index("c")
        sid = lax.axis_index("s")

        zeros16 = jnp.zeros((16,), jnp.float32)

        def zrow(i, carry):
            zbuf[i, pl.ds(0, 16)] = zeros16
            zbuf[i, pl.ds(8, 16)] = zeros16
            return carry

        lax.fori_loop(0, _SZROWS, zrow, 0)

        for p in range(2):
            grp = cid * 2 + p
            for t in range(_NPT // _SZROWS):
                pltpu.sync_copy(
                    zbuf, acc.at[pl.ds(sid * _NPT + t * _SZROWS, _SZROWS)])
            plsc.subcore_barrier()

            def chunk(t, carry):
                row0 = sid * _SROWS_PER_TILE + t * _SCHUNK_ROWS
                ebase = row0 * 128
                pltpu.sync_copy(
                    msgs2_hbm.at[pl.ds(ebase, _SCHUNK),
                                 pl.ds(grp * _SG, _SG)], dbuf)
                pltpu.sync_copy(dst2d_hbm.at[pl.ds(row0, _SCHUNK_ROWS)], ibuf)
                for j in range(_SCHUNK_ROWS):
                    pltpu.sync_copy(dbuf.at[pl.ds(j * 128, 128)],
                                    acc.at[ibuf.at[j]], add=True)
                return carry

            lax.fori_loop(0, _SNCHUNK, chunk, 0)
            plsc.subcore_barrier()

            pltpu.sync_copy(acc.at[pl.ds(sid * _NPT, _NPT)],
                            out_hbm.at[grp, pl.ds(sid * _NPT, _NPT)])

    return k(dst2d, msgs2)


_EB = 2048
_ENB = E // _EB


def _edge_body(cs_ref, cd_ref, ef_ref, emb_ref, mW0_ref, b0_ref, mW1_ref,
               b1_ref, mW2p_ref, b2p_ref, out_ref):
    f32 = jnp.float32
    dims11 = (((1,), (1,)), ((), ()))
    dims10 = (((1,), (0,)), ((), ()))
    cs = cs_ref[0, 0, :]
    cd = cd_ref[0, 0, :]
    iota10 = lax.broadcasted_iota(jnp.int32, (1, 10), 1)
    oh_s = (cs[:, None] == iota10).astype(f32)
    oh_d = (cd[:, None] == iota10).astype(f32)
    emb = emb_ref[...]
    TA = lax.dot_general(emb, mW0_ref[:, 0:EMB], dims11,
                         preferred_element_type=f32)
    TB = lax.dot_general(emb, mW0_ref[:, EMB:2 * EMB], dims11,
                         preferred_element_type=f32)
    h0 = (lax.dot_general(oh_s, TA, dims10, preferred_element_type=f32)
          + lax.dot_general(oh_d, TB, dims10, preferred_element_type=f32)
          + lax.dot_general(ef_ref[...], mW0_ref[:, 2 * EMB:], dims11,
                            preferred_element_type=f32)
          + b0_ref[...])
    h1 = jnp.maximum(h0, 0.0)
    h2 = jnp.maximum(
        lax.dot_general(h1, mW1_ref[...], dims11, preferred_element_type=f32)
        + b1_ref[...], 0.0)
    out_ref[...] = lax.dot_general(h2, mW2p_ref[...], dims11,
                                   preferred_element_type=f32) + b2p_ref[...]


def _edge_mlp(cs3, cd3, ef, emb_s, mW0, b0r, mW1, b1r, mW2p, b2pr):
    full = lambda shape: pl.BlockSpec(shape, lambda i, _s=shape: tuple(0 for _ in _s))
    return pl.pallas_call(
        _edge_body,
        grid=(_ENB,),
        in_specs=[
            pl.BlockSpec((1, 1, _EB), lambda i: (i, 0, 0)),
            pl.BlockSpec((1, 1, _EB), lambda i: (i, 0, 0)),
            pl.BlockSpec((_EB, DE), lambda i: (i, 0)),
            full((10, EMB)),
            full((H, 2 * EMB + DE)),
            full((1, H)),
            full((H, H)),
            full((1, H)),
            full((128, H)),
            full((1, 128)),
        ],
        out_specs=pl.BlockSpec((_EB, 128), lambda i: (i, 0)),
        out_shape=jax.ShapeDtypeStruct((E, 128), jnp.float32),
        compiler_params=pltpu.CompilerParams(
            dimension_semantics=("arbitrary",)),
    )(cs3, cd3, ef, emb_s, mW0, b0r, mW1, b1r, mW2p, b2pr)


_RB = 1296
_RNB = N // _RB


def _post_body(agg_ref, pz_ref, sh_ref, sc_ref, pW0_ref, pb0_ref, pW1_ref,
               pb1_ref, pW2_ref, pb2_ref, Wi_ref, Wf_ref, Wg_ref, Wo_ref,
               Ui_ref, Uf_ref, Ug_ref, Uo_ref, bi_ref, bf_ref, bg_ref,
               bo_ref, oW_ref, ob_ref, h_ref, c_ref, o_ref):
    f32 = jnp.float32
    dims11 = (((1,), (1,)), ((), ()))

    def dot(x, w):
        return lax.dot_general(x, w, dims11, preferred_element_type=f32)

    g0 = (dot(agg_ref[0], pW0_ref[:, 0:_SG])
          + dot(agg_ref[1], pW0_ref[:, _SG:2 * _SG])
          + dot(agg_ref[2], pW0_ref[:, 2 * _SG:3 * _SG])
          + dot(agg_ref[3], pW0_ref[:, 3 * _SG:4 * _SG])
          + dot(pz_ref[...], pW0_ref[:, 96:112]) + pb0_ref[...])
    h = jnp.maximum(g0, 0.0)
    h = jnp.maximum(dot(h, pW1_ref[...]) + pb1_ref[...], 0.0)
    hp = dot(h, pW2_ref[...]) + pb2_ref[...]
    sh = sh_ref[...]
    ii = jax.nn.sigmoid(dot(hp, Wi_ref[...]) + dot(sh, Ui_ref[...]) + bi_ref[...])
    ff = jax.nn.sigmoid(dot(hp, Wf_ref[...]) + dot(sh, Uf_ref[...]) + bf_ref[...])
    gg = jnp.tanh(dot(hp, Wg_ref[...]) + dot(sh, Ug_ref[...]) + bg_ref[...])
    oo = jax.nn.sigmoid(dot(hp, Wo_ref[...]) + dot(sh, Uo_ref[...]) + bo_ref[...])
    cn = ff * sc_ref[...] + ii * gg
    hn = oo * jnp.tanh(cn)
    h_ref[...] = hn
    c_ref[...] = cn
    o_ref[...] = dot(hn, oW_ref[...]) + ob_ref[...]


def _post(agg, puzzle, sh, sc, pW0, pb0r, pW1, pb1r, pW2, pb2r, Ws, Us, bs,
          oW, obr):
    full = lambda shape: pl.BlockSpec(shape, lambda i, _s=shape: tuple(0 for _ in _s))
    return pl.pallas_call(
        _post_body,
        grid=(_RNB,),
        in_specs=[
            pl.BlockSpec((4, _RB, _SG), lambda i: (0, i, 0)),
            pl.BlockSpec((_RB, EMB), lambda i: (i, 0)),
            pl.BlockSpec((_RB, H), lambda i: (i, 0)),
            pl.BlockSpec((_RB, H), lambda i: (i, 0)),
            full((H, H + EMB)),
            full((1, H)),
            full((H, H)),
            full((1, H)),
            full((H, H)),
            full((1, H)),
            *[full((H, H)) for _ in range(8)],
            *[full((1, H)) for _ in range(4)],
            full((10, H)),
            full((1, 10)),
        ],
        out_specs=[
            pl.BlockSpec((_RB, H), lambda i: (i, 0)),
            pl.BlockSpec((_RB, H), lambda i: (i, 0)),
            pl.BlockSpec((_RB, 10), lambda i: (i, 0)),
        ],
        out_shape=[
            jax.ShapeDtypeStruct((N, H), jnp.float32),
            jax.ShapeDtypeStruct((N, H), jnp.float32),
            jax.ShapeDtypeStruct((N, 10), jnp.float32),
        ],
        compiler_params=pltpu.CompilerParams(
            dimension_semantics=("arbitrary",)),
    )(agg, puzzle, sh, sc, pW0, pb0r, pW1, pb1r, pW2, pb2r, *Ws, *Us, *bs,
      oW, obr)


def kernel(puzzle, nodes, edges, edge_features, state_h, state_c, first, emb,
           mW0, mb0, mW1, mb1, mW2, mb2, pW0, pb0, pW1, pb1, pW2, pb2, W_ih,
           W_hh, b_ih, b_hh, oW, ob):
    f32 = jnp.float32
    nodes = nodes.astype(jnp.int32)
    eflat = edges.astype(jnp.int32).reshape(2 * E)
    dst2d = edges[1].astype(jnp.int32).reshape(E // 128, 128)

    cls = _gather_cls(nodes, eflat)
    cs3 = cls[:E].reshape(_ENB, 1, _EB)
    cd3 = cls[E:].reshape(_ENB, 1, _EB)

    emb_s = emb * jnp.asarray(first, f32)
    mW2p = jnp.concatenate([mW2, jnp.zeros((128 - H, H), f32)], axis=0)
    b2p = jnp.concatenate([mb2, jnp.zeros((128 - H,), f32)]).reshape(1, 128)
    msgs2 = _edge_mlp(
        cs3, cd3, edge_features, emb_s, mW0, mb0.reshape(1, H), mW1,
        mb1.reshape(1, H), mW2p, b2p)

    agg = _scatter_agg(dst2d, msgs2)

    b = (b_ih + b_hh).reshape(1, 4 * H)
    Ws = [W_ih[i * H:(i + 1) * H] for i in range(4)]
    Us = [W_hh[i * H:(i + 1) * H] for i in range(4)]
    bs = [b[:, i * H:(i + 1) * H] for i in range(4)]
    h_new, c_new, out10 = _post(
        agg, puzzle, state_h, state_c, pW0, pb0.reshape(1, H), pW1,
        pb1.reshape(1, H), pW2, pb2.reshape(1, H), Ws, Us, bs, oW,
        ob.reshape(1, 10))
    return (h_new, c_new, out10.reshape(-1, 81, 10))


# speedup vs baseline: 6.9904x; 1.1747x over previous
import functools

import jax
import jax.numpy as jnp
from jax import lax
from jax.experimental import pallas as pl
from jax.experimental.pallas import tpu as pltpu
from jax.experimental.pallas import tpu_sc as plsc

N = 41472
E = 829440
H = 96
EMB = 16
DE = 16

_GW = 32
_GCHUNK = (2 * E) // _GW
_GSUB = 6480
_GNSUB = _GCHUNK // _GSUB


def _gather_cls(nodes, eflat):
    mesh = plsc.VectorSubcoreMesh(core_axis_name="c", subcore_axis_name="s")

    @functools.partial(
        pl.kernel,
        out_type=jax.ShapeDtypeStruct((2 * E,), jnp.int32),
        mesh=mesh,
        scratch_types=[
            pltpu.VMEM_SHARED((N,), jnp.int32),
            pltpu.VMEM((_GSUB,), jnp.int32),
            pltpu.VMEM((_GSUB,), jnp.int32),
            pltpu.VMEM((_GSUB,), jnp.int32),
            pltpu.VMEM((_GSUB,), jnp.int32),
            pltpu.SemaphoreType.DMA((2,)),
            pltpu.SemaphoreType.DMA,
        ],
    )
    def k(nodes_hbm, eflat_hbm, out_hbm, tbl, ibuf0, ibuf1, obuf0, obuf1,
          isem, gsem):
        ibufs = (ibuf0, ibuf1)
        obufs = (obuf0, obuf1)
        cid = lax.axis_index("c")
        sid = lax.axis_index("s")
        wid = sid * 2 + cid
        base = wid * _GCHUNK

        @pl.when(sid == 0)
        def _():
            pltpu.sync_copy(nodes_hbm, tbl)
        plsc.subcore_barrier()

        pltpu.async_copy(eflat_hbm.at[pl.ds(base, _GSUB)], ibufs[0],
                         isem.at[0])
        for sc in range(_GNSUB):
            b = sc % 2
            off = base + sc * _GSUB
            if sc + 1 < _GNSUB:
                pltpu.async_copy(
                    eflat_hbm.at[pl.ds(off + _GSUB, _GSUB)],
                    ibufs[1 - b], isem.at[1 - b])
            pltpu.make_async_copy(eflat_hbm.at[pl.ds(off, _GSUB)],
                                  ibufs[b], isem.at[b]).wait()
            pltpu.async_copy(tbl.at[ibufs[b]], obufs[b], gsem).wait()
            pltpu.sync_copy(obufs[b], out_hbm.at[pl.ds(off, _GSUB)])

    return k(nodes, eflat)


_SG = 24
_SROWS_PER_TILE = (E // 128) // 16
_SCHUNK_ROWS = 9
_SCHUNK = _SCHUNK_ROWS * 128
_SNCHUNK = _SROWS_PER_TILE // _SCHUNK_ROWS
_SZROWS = 162
_NPT = N // 16


def _scatter_agg(dst2d, msgs2):
    mesh = plsc.VectorSubcoreMesh(core_axis_name="c", subcore_axis_name="s")

    @functools.partial(
        pl.kernel,
        out_type=jax.ShapeDtypeStruct((4, N, _SG), jnp.float32),
        mesh=mesh,
        scratch_types=[
            pltpu.VMEM_SHARED((N, _SG), jnp.float32),
            pltpu.VMEM((_SZROWS, _SG), jnp.float32),
            pltpu.VMEM((2, _SCHUNK, _SG), jnp.float32),
            pltpu.VMEM((2, _SCHUNK_ROWS, 128), jnp.int32),
            pltpu.SemaphoreType.DMA((2,)),
            pltpu.SemaphoreType.DMA((2,)),
            pltpu.SemaphoreType.DMA,
        ],
        compiler_params=pltpu.CompilerParams(use_tc_tiling_on_sc=False),
    )
    def k(dst2d_hbm, msgs2_hbm, out_hbm, acc, zbuf, dbuf, ibuf, dsem, isem,
          ssem):
        cid = lax.axis_index("c")
        sid = lax.axis_index("s")

        zeros16 = jnp.zeros((16,), jnp.float32)

        def zrow(i, carry):
            zbuf[i, pl.ds(0, 16)] = zeros16
            zbuf[i, pl.ds(8, 16)] = zeros16
            return carry

        lax.fori_loop(0, _SZROWS, zrow, 0)

        for p in range(2):
            grp = cid * 2 + p
            col0 = grp * _SG
            for t in range(_NPT // _SZROWS):
                pltpu.sync_copy(
                    zbuf, acc.at[pl.ds(sid * _NPT + t * _SZROWS, _SZROWS)])
            plsc.subcore_barrier()

            def start_in(t, b):
                row0 = sid * _SROWS_PER_TILE + t * _SCHUNK_ROWS
                pltpu.async_copy(
                    msgs2_hbm.at[pl.ds(row0 * 128, _SCHUNK),
                                 pl.ds(col0, _SG)],
                    dbuf.at[b], dsem.at[b])
                pltpu.async_copy(dst2d_hbm.at[pl.ds(row0, _SCHUNK_ROWS)],
                                 ibuf.at[b], isem.at[b])

            def wait_in(t, b):
                row0 = sid * _SROWS_PER_TILE + t * _SCHUNK_ROWS
                pltpu.make_async_copy(
                    msgs2_hbm.at[pl.ds(row0 * 128, _SCHUNK),
                                 pl.ds(col0, _SG)],
                    dbuf.at[b], dsem.at[b]).wait()
                pltpu.make_async_copy(
                    dst2d_hbm.at[pl.ds(row0, _SCHUNK_ROWS)],
                    ibuf.at[b], isem.at[b]).wait()

            start_in(0, 0)

            def chunk2(t2, carry):
                for b in range(2):
                    t = t2 * 2 + b

                    @pl.when(t < _SNCHUNK)
                    def _():
                        @pl.when(t + 1 < _SNCHUNK)
                        def _():
                            start_in(t + 1, 1 - b)
                        wait_in(t, b)
                        descs = []
                        for j in range(_SCHUNK_ROWS):
                            descs.append(pltpu.async_copy(
                                dbuf.at[b, pl.ds(j * 128, 128)],
                                acc.at[ibuf.at[b, j]], ssem, add=True))
                        for d in descs:
                            d.wait()
                return carry

            lax.fori_loop(0, (_SNCHUNK + 1) // 2, chunk2, 0)
            plsc.subcore_barrier()

            pltpu.sync_copy(acc.at[pl.ds(sid * _NPT, _NPT)],
                            out_hbm.at[grp, pl.ds(sid * _NPT, _NPT)])

    return k(dst2d, msgs2)


_EB = 2048
_ENB = E // _EB


def _edge_body(cs_ref, cd_ref, ef_ref, emb_ref, mW0_ref, b0_ref, mW1_ref,
               b1_ref, mW2p_ref, b2p_ref, out_ref):
    f32 = jnp.float32
    dims11 = (((1,), (1,)), ((), ()))
    dims10 = (((1,), (0,)), ((), ()))
    cs = cs_ref[0, 0, :]
    cd = cd_ref[0, 0, :]
    iota10 = lax.broadcasted_iota(jnp.int32, (1, 10), 1)
    oh_s = (cs[:, None] == iota10).astype(f32)
    oh_d = (cd[:, None] == iota10).astype(f32)
    emb = emb_ref[...]
    TA = lax.dot_general(emb, mW0_ref[:, 0:EMB], dims11,
                         preferred_element_type=f32)
    TB = lax.dot_general(emb, mW0_ref[:, EMB:2 * EMB], dims11,
                         preferred_element_type=f32)
    h0 = (lax.dot_general(oh_s, TA, dims10, preferred_element_type=f32)
          + lax.dot_general(oh_d, TB, dims10, preferred_element_type=f32)
          + lax.dot_general(ef_ref[...], mW0_ref[:, 2 * EMB:], dims11,
                            preferred_element_type=f32)
          + b0_ref[...])
    h1 = jnp.maximum(h0, 0.0)
    h2 = jnp.maximum(
        lax.dot_general(h1, mW1_ref[...], dims11, preferred_element_type=f32)
        + b1_ref[...], 0.0)
    out_ref[...] = lax.dot_general(h2, mW2p_ref[...], dims11,
                                   preferred_element_type=f32) + b2p_ref[...]


def _edge_mlp(cs3, cd3, ef, emb_s, mW0, b0r, mW1, b1r, mW2p, b2pr):
    full = lambda shape: pl.BlockSpec(shape, lambda i, _s=shape: tuple(0 for _ in _s))
    return pl.pallas_call(
        _edge_body,
        grid=(_ENB,),
        in_specs=[
            pl.BlockSpec((1, 1, _EB), lambda i: (i, 0, 0)),
            pl.BlockSpec((1, 1, _EB), lambda i: (i, 0, 0)),
            pl.BlockSpec((_EB, DE), lambda i: (i, 0)),
            full((10, EMB)),
            full((H, 2 * EMB + DE)),
            full((1, H)),
            full((H, H)),
            full((1, H)),
            full((128, H)),
            full((1, 128)),
        ],
        out_specs=pl.BlockSpec((_EB, 128), lambda i: (i, 0)),
        out_shape=jax.ShapeDtypeStruct((E, 128), jnp.float32),
        compiler_params=pltpu.CompilerParams(
            dimension_semantics=("arbitrary",)),
    )(cs3, cd3, ef, emb_s, mW0, b0r, mW1, b1r, mW2p, b2pr)


_RB = 1296
_RNB = N // _RB


def _post_body(agg_ref, pz_ref, sh_ref, sc_ref, pW0_ref, pb0_ref, pW1_ref,
               pb1_ref, pW2_ref, pb2_ref, Wi_ref, Wf_ref, Wg_ref, Wo_ref,
               Ui_ref, Uf_ref, Ug_ref, Uo_ref, bi_ref, bf_ref, bg_ref,
               bo_ref, oW_ref, ob_ref, h_ref, c_ref, o_ref):
    f32 = jnp.float32
    dims11 = (((1,), (1,)), ((), ()))

    def dot(x, w):
        return lax.dot_general(x, w, dims11, preferred_element_type=f32)

    g0 = (dot(agg_ref[0], pW0_ref[:, 0:_SG])
          + dot(agg_ref[1], pW0_ref[:, _SG:2 * _SG])
          + dot(agg_ref[2], pW0_ref[:, 2 * _SG:3 * _SG])
          + dot(agg_ref[3], pW0_ref[:, 3 * _SG:4 * _SG])
          + dot(pz_ref[...], pW0_ref[:, 96:112]) + pb0_ref[...])
    h = jnp.maximum(g0, 0.0)
    h = jnp.maximum(dot(h, pW1_ref[...]) + pb1_ref[...], 0.0)
    hp = dot(h, pW2_ref[...]) + pb2_ref[...]
    sh = sh_ref[...]
    ii = jax.nn.sigmoid(dot(hp, Wi_ref[...]) + dot(sh, Ui_ref[...]) + bi_ref[...])
    ff = jax.nn.sigmoid(dot(hp, Wf_ref[...]) + dot(sh, Uf_ref[...]) + bf_ref[...])
    gg = jnp.tanh(dot(hp, Wg_ref[...]) + dot(sh, Ug_ref[...]) + bg_ref[...])
    oo = jax.nn.sigmoid(dot(hp, Wo_ref[...]) + dot(sh, Uo_ref[...]) + bo_ref[...])
    cn = ff * sc_ref[...] + ii * gg
    hn = oo * jnp.tanh(cn)
    h_ref[...] = hn
    c_ref[...] = cn
    o_ref[...] = dot(hn, oW_ref[...]) + ob_ref[...]


def _post(agg, puzzle, sh, sc, pW0, pb0r, pW1, pb1r, pW2, pb2r, Ws, Us, bs,
          oW, obr):
    full = lambda shape: pl.BlockSpec(shape, lambda i, _s=shape: tuple(0 for _ in _s))
    return pl.pallas_call(
        _post_body,
        grid=(_RNB,),
        in_specs=[
            pl.BlockSpec((4, _RB, _SG), lambda i: (0, i, 0)),
            pl.BlockSpec((_RB, EMB), lambda i: (i, 0)),
            pl.BlockSpec((_RB, H), lambda i: (i, 0)),
            pl.BlockSpec((_RB, H), lambda i: (i, 0)),
            full((H, H + EMB)),
            full((1, H)),
            full((H, H)),
            full((1, H)),
            full((H, H)),
            full((1, H)),
            *[full((H, H)) for _ in range(8)],
            *[full((1, H)) for _ in range(4)],
            full((10, H)),
            full((1, 10)),
        ],
        out_specs=[
            pl.BlockSpec((_RB, H), lambda i: (i, 0)),
            pl.BlockSpec((_RB, H), lambda i: (i, 0)),
            pl.BlockSpec((_RB, 10), lambda i: (i, 0)),
        ],
        out_shape=[
            jax.ShapeDtypeStruct((N, H), jnp.float32),
            jax.ShapeDtypeStruct((N, H), jnp.float32),
            jax.ShapeDtypeStruct((N, 10), jnp.float32),
        ],
        compiler_params=pltpu.CompilerParams(
            dimension_semantics=("arbitrary",)),
    )(agg, puzzle, sh, sc, pW0, pb0r, pW1, pb1r, pW2, pb2r, *Ws, *Us, *bs,
      oW, obr)


def kernel(puzzle, nodes, edges, edge_features, state_h, state_c, first, emb,
           mW0, mb0, mW1, mb1, mW2, mb2, pW0, pb0, pW1, pb1, pW2, pb2, W_ih,
           W_hh, b_ih, b_hh, oW, ob):
    f32 = jnp.float32
    nodes = nodes.astype(jnp.int32)
    eflat = edges.astype(jnp.int32).reshape(2 * E)
    dst2d = edges[1].astype(jnp.int32).reshape(E // 128, 128)

    cls = _gather_cls(nodes, eflat)
    cs3 = cls[:E].reshape(_ENB, 1, _EB)
    cd3 = cls[E:].reshape(_ENB, 1, _EB)

    emb_s = emb * jnp.asarray(first, f32)
    mW2p = jnp.concatenate([mW2, jnp.zeros((128 - H, H), f32)], axis=0)
    b2p = jnp.concatenate([mb2, jnp.zeros((128 - H,), f32)]).reshape(1, 128)
    msgs2 = _edge_mlp(
        cs3, cd3, edge_features, emb_s, mW0, mb0.reshape(1, H), mW1,
        mb1.reshape(1, H), mW2p, b2p)

    agg = _scatter_agg(dst2d, msgs2)

    b = (b_ih + b_hh).reshape(1, 4 * H)
    Ws = [W_ih[i * H:(i + 1) * H] for i in range(4)]
    Us = [W_hh[i * H:(i + 1) * H] for i in range(4)]
    bs = [b[:, i * H:(i + 1) * H] for i in range(4)]
    h_new, c_new, out10 = _post(
        agg, puzzle, state_h, state_c, pW0, pb0.reshape(1, H), pW1,
        pb1.reshape(1, H), pW2, pb2.reshape(1, H), Ws, Us, bs, oW,
        ob.reshape(1, 10))
    return (h_new, c_new, out10.reshape(-1, 81, 10))


# speedup vs baseline: 7.9509x; 1.1374x over previous
import functools

import jax
import jax.numpy as jnp
from jax import lax
from jax.experimental import pallas as pl
from jax.experimental.pallas import tpu as pltpu
from jax.experimental.pallas import tpu_sc as plsc

N = 41472
E = 829440
H = 96
EMB = 16
DE = 16

_GW = 32
_GCHUNK = (2 * E) // _GW
_GSUB = 6480
_GNSUB = _GCHUNK // _GSUB


def _gather_cls(nodes, eflat):
    mesh = plsc.VectorSubcoreMesh(core_axis_name="c", subcore_axis_name="s")

    @functools.partial(
        pl.kernel,
        out_type=jax.ShapeDtypeStruct((2 * E,), jnp.int32),
        mesh=mesh,
        scratch_types=[
            pltpu.VMEM_SHARED((N,), jnp.int32),
            pltpu.VMEM((_GSUB,), jnp.int32),
            pltpu.VMEM((_GSUB,), jnp.int32),
            pltpu.VMEM((_GSUB,), jnp.int32),
            pltpu.VMEM((_GSUB,), jnp.int32),
            pltpu.SemaphoreType.DMA((2,)),
            pltpu.SemaphoreType.DMA,
        ],
    )
    def k(nodes_hbm, eflat_hbm, out_hbm, tbl, ibuf0, ibuf1, obuf0, obuf1,
          isem, gsem):
        ibufs = (ibuf0, ibuf1)
        obufs = (obuf0, obuf1)
        cid = lax.axis_index("c")
        sid = lax.axis_index("s")
        wid = sid * 2 + cid
        base = wid * _GCHUNK

        @pl.when(sid == 0)
        def _():
            pltpu.sync_copy(nodes_hbm, tbl)
        plsc.subcore_barrier()

        pltpu.async_copy(eflat_hbm.at[pl.ds(base, _GSUB)], ibufs[0],
                         isem.at[0])
        for sc in range(_GNSUB):
            b = sc % 2
            off = base + sc * _GSUB
            if sc + 1 < _GNSUB:
                pltpu.async_copy(
                    eflat_hbm.at[pl.ds(off + _GSUB, _GSUB)],
                    ibufs[1 - b], isem.at[1 - b])
            pltpu.make_async_copy(eflat_hbm.at[pl.ds(off, _GSUB)],
                                  ibufs[b], isem.at[b]).wait()
            pltpu.async_copy(tbl.at[ibufs[b]], obufs[b], gsem).wait()
            pltpu.sync_copy(obufs[b], out_hbm.at[pl.ds(off, _GSUB)])

    return k(nodes, eflat)


_SG = 24
_SROWS_PER_TILE = (E // 128) // 16
_SCHUNK_ROWS = 9
_SCHUNK = _SCHUNK_ROWS * 128
_SNCHUNK = _SROWS_PER_TILE // _SCHUNK_ROWS
_SZROWS = 162
_NPT = N // 16


def _scatter_agg(dst2d, msgs2):
    mesh = plsc.VectorSubcoreMesh(core_axis_name="c", subcore_axis_name="s")

    @functools.partial(
        pl.kernel,
        out_type=jax.ShapeDtypeStruct((4, N, _SG), jnp.float32),
        mesh=mesh,
        scratch_types=[
            pltpu.VMEM_SHARED((N, _SG), jnp.float32),
            pltpu.VMEM((_SZROWS, _SG), jnp.float32),
            pltpu.VMEM((2, _SCHUNK, _SG), jnp.float32),
            pltpu.VMEM((2, _SCHUNK_ROWS, 128), jnp.int32),
            pltpu.SemaphoreType.DMA((2,)),
            pltpu.SemaphoreType.DMA((2,)),
            pltpu.SemaphoreType.DMA,
        ],
        compiler_params=pltpu.CompilerParams(use_tc_tiling_on_sc=False),
    )
    def k(dst2d_hbm, msgs2_hbm, out_hbm, acc, zbuf, dbuf, ibuf, dsem, isem,
          ssem):
        cid = lax.axis_index("c")
        sid = lax.axis_index("s")

        zeros16 = jnp.zeros((16,), jnp.float32)

        def zrow(i, carry):
            zbuf[i, pl.ds(0, 16)] = zeros16
            zbuf[i, pl.ds(8, 16)] = zeros16
            return carry

        lax.fori_loop(0, _SZROWS, zrow, 0)

        for p in range(2):
            grp = cid * 2 + p
            col0 = grp * _SG
            for t in range(_NPT // _SZROWS):
                pltpu.sync_copy(
                    zbuf, acc.at[pl.ds(sid * _NPT + t * _SZROWS, _SZROWS)])
            plsc.subcore_barrier()

            def start_in(t, b):
                row0 = sid * _SROWS_PER_TILE + t * _SCHUNK_ROWS
                pltpu.async_copy(
                    msgs2_hbm.at[pl.ds(row0 * 128, _SCHUNK),
                                 pl.ds(col0, _SG)],
                    dbuf.at[b], dsem.at[b])
                pltpu.async_copy(dst2d_hbm.at[pl.ds(row0, _SCHUNK_ROWS)],
                                 ibuf.at[b], isem.at[b])

            def wait_in(t, b):
                row0 = sid * _SROWS_PER_TILE + t * _SCHUNK_ROWS
                pltpu.make_async_copy(
                    msgs2_hbm.at[pl.ds(row0 * 128, _SCHUNK),
                                 pl.ds(col0, _SG)],
                    dbuf.at[b], dsem.at[b]).wait()
                pltpu.make_async_copy(
                    dst2d_hbm.at[pl.ds(row0, _SCHUNK_ROWS)],
                    ibuf.at[b], isem.at[b]).wait()

            start_in(0, 0)

            def chunk2(t2, carry):
                for b in range(2):
                    t = t2 * 2 + b

                    @pl.when(t < _SNCHUNK)
                    def _():
                        @pl.when(t + 1 < _SNCHUNK)
                        def _():
                            start_in(t + 1, 1 - b)
                        wait_in(t, b)
                        descs = []
                        for j in range(_SCHUNK_ROWS):
                            descs.append(pltpu.async_copy(
                                dbuf.at[b, pl.ds(j * 128, 128)],
                                acc.at[ibuf.at[b, j]], ssem, add=True))
                        for d in descs:
                            d.wait()
                return carry

            lax.fori_loop(0, (_SNCHUNK + 1) // 2, chunk2, 0)
            plsc.subcore_barrier()

            pltpu.sync_copy(acc.at[pl.ds(sid * _NPT, _NPT)],
                            out_hbm.at[grp, pl.ds(sid * _NPT, _NPT)])

    return k(dst2d, msgs2)


_EB = 5120
_ENB = E // _EB


def _edge_body(cs_ref, cd_ref, ef_ref, emb_ref, mW0_ref, b0_ref, mW1_ref,
               b1_ref, mW2p_ref, b2p_ref, out_ref):
    f32 = jnp.float32
    dims11 = (((1,), (1,)), ((), ()))
    dims10 = (((1,), (0,)), ((), ()))
    cs = cs_ref[0, 0, :]
    cd = cd_ref[0, 0, :]
    iota10 = lax.broadcasted_iota(jnp.int32, (1, 10), 1)
    oh_s = (cs[:, None] == iota10).astype(f32)
    oh_d = (cd[:, None] == iota10).astype(f32)
    emb = emb_ref[...]
    TA = lax.dot_general(emb, mW0_ref[:, 0:EMB], dims11,
                         preferred_element_type=f32)
    TB = lax.dot_general(emb, mW0_ref[:, EMB:2 * EMB], dims11,
                         preferred_element_type=f32)
    h0 = (lax.dot_general(oh_s, TA, dims10, preferred_element_type=f32)
          + lax.dot_general(oh_d, TB, dims10, preferred_element_type=f32)
          + lax.dot_general(ef_ref[...], mW0_ref[:, 2 * EMB:], dims11,
                            preferred_element_type=f32)
          + b0_ref[...])
    bf16 = jnp.bfloat16
    h1 = jnp.maximum(h0, 0.0).astype(bf16)
    h2 = jnp.maximum(
        lax.dot_general(h1, mW1_ref[...].astype(bf16), dims11,
                        preferred_element_type=f32)
        + b1_ref[...], 0.0).astype(bf16)
    out_ref[...] = lax.dot_general(h2, mW2p_ref[...].astype(bf16), dims11,
                                   preferred_element_type=f32) + b2p_ref[...]


def _edge_mlp(cs3, cd3, ef, emb_s, mW0, b0r, mW1, b1r, mW2p, b2pr):
    full = lambda shape: pl.BlockSpec(shape, lambda i, _s=shape: tuple(0 for _ in _s))
    return pl.pallas_call(
        _edge_body,
        grid=(_ENB,),
        in_specs=[
            pl.BlockSpec((1, 1, _EB), lambda i: (i, 0, 0)),
            pl.BlockSpec((1, 1, _EB), lambda i: (i, 0, 0)),
            pl.BlockSpec((_EB, DE), lambda i: (i, 0)),
            full((10, EMB)),
            full((H, 2 * EMB + DE)),
            full((1, H)),
            full((H, H)),
            full((1, H)),
            full((128, H)),
            full((1, 128)),
        ],
        out_specs=pl.BlockSpec((_EB, 128), lambda i: (i, 0)),
        out_shape=jax.ShapeDtypeStruct((E, 128), jnp.float32),
        compiler_params=pltpu.CompilerParams(
            dimension_semantics=("arbitrary",)),
    )(cs3, cd3, ef, emb_s, mW0, b0r, mW1, b1r, mW2p, b2pr)


_RB = 1296
_RNB = N // _RB


def _post_body(agg_ref, pz_ref, sh_ref, sc_ref, pW0_ref, pb0_ref, pW1_ref,
               pb1_ref, pW2_ref, pb2_ref, Wi_ref, Wf_ref, Wg_ref, Wo_ref,
               Ui_ref, Uf_ref, Ug_ref, Uo_ref, bi_ref, bf_ref, bg_ref,
               bo_ref, oW_ref, ob_ref, h_ref, c_ref, o_ref):
    f32 = jnp.float32
    dims11 = (((1,), (1,)), ((), ()))

    def dot(x, w):
        return lax.dot_general(x, w, dims11, preferred_element_type=f32)

    g0 = (dot(agg_ref[0], pW0_ref[:, 0:_SG])
          + dot(agg_ref[1], pW0_ref[:, _SG:2 * _SG])
          + dot(agg_ref[2], pW0_ref[:, 2 * _SG:3 * _SG])
          + dot(agg_ref[3], pW0_ref[:, 3 * _SG:4 * _SG])
          + dot(pz_ref[...], pW0_ref[:, 96:112]) + pb0_ref[...])
    h = jnp.maximum(g0, 0.0)
    h = jnp.maximum(dot(h, pW1_ref[...]) + pb1_ref[...], 0.0)
    hp = dot(h, pW2_ref[...]) + pb2_ref[...]
    sh = sh_ref[...]
    ii = jax.nn.sigmoid(dot(hp, Wi_ref[...]) + dot(sh, Ui_ref[...]) + bi_ref[...])
    ff = jax.nn.sigmoid(dot(hp, Wf_ref[...]) + dot(sh, Uf_ref[...]) + bf_ref[...])
    gg = jnp.tanh(dot(hp, Wg_ref[...]) + dot(sh, Ug_ref[...]) + bg_ref[...])
    oo = jax.nn.sigmoid(dot(hp, Wo_ref[...]) + dot(sh, Uo_ref[...]) + bo_ref[...])
    cn = ff * sc_ref[...] + ii * gg
    hn = oo * jnp.tanh(cn)
    h_ref[...] = hn
    c_ref[...] = cn
    o_ref[...] = dot(hn, oW_ref[...]) + ob_ref[...]


def _post(agg, puzzle, sh, sc, pW0, pb0r, pW1, pb1r, pW2, pb2r, Ws, Us, bs,
          oW, obr):
    full = lambda shape: pl.BlockSpec(shape, lambda i, _s=shape: tuple(0 for _ in _s))
    return pl.pallas_call(
        _post_body,
        grid=(_RNB,),
        in_specs=[
            pl.BlockSpec((4, _RB, _SG), lambda i: (0, i, 0)),
            pl.BlockSpec((_RB, EMB), lambda i: (i, 0)),
            pl.BlockSpec((_RB, H), lambda i: (i, 0)),
            pl.BlockSpec((_RB, H), lambda i: (i, 0)),
            full((H, H + EMB)),
            full((1, H)),
            full((H, H)),
            full((1, H)),
            full((H, H)),
            full((1, H)),
            *[full((H, H)) for _ in range(8)],
            *[full((1, H)) for _ in range(4)],
            full((10, H)),
            full((1, 10)),
        ],
        out_specs=[
            pl.BlockSpec((_RB, H), lambda i: (i, 0)),
            pl.BlockSpec((_RB, H), lambda i: (i, 0)),
            pl.BlockSpec((_RB, 10), lambda i: (i, 0)),
        ],
        out_shape=[
            jax.ShapeDtypeStruct((N, H), jnp.float32),
            jax.ShapeDtypeStruct((N, H), jnp.float32),
            jax.ShapeDtypeStruct((N, 10), jnp.float32),
        ],
        compiler_params=pltpu.CompilerParams(
            dimension_semantics=("arbitrary",)),
    )(agg, puzzle, sh, sc, pW0, pb0r, pW1, pb1r, pW2, pb2r, *Ws, *Us, *bs,
      oW, obr)


def kernel(puzzle, nodes, edges, edge_features, state_h, state_c, first, emb,
           mW0, mb0, mW1, mb1, mW2, mb2, pW0, pb0, pW1, pb1, pW2, pb2, W_ih,
           W_hh, b_ih, b_hh, oW, ob):
    f32 = jnp.float32
    nodes = nodes.astype(jnp.int32)
    eflat = edges.astype(jnp.int32).reshape(2 * E)
    dst2d = edges[1].astype(jnp.int32).reshape(E // 128, 128)

    cls = _gather_cls(nodes, eflat)
    cs3 = cls[:E].reshape(_ENB, 1, _EB)
    cd3 = cls[E:].reshape(_ENB, 1, _EB)

    emb_s = emb * jnp.asarray(first, f32)
    mW2p = jnp.concatenate([mW2, jnp.zeros((128 - H, H), f32)], axis=0)
    b2p = jnp.concatenate([mb2, jnp.zeros((128 - H,), f32)]).reshape(1, 128)
    msgs2 = _edge_mlp(
        cs3, cd3, edge_features, emb_s, mW0, mb0.reshape(1, H), mW1,
        mb1.reshape(1, H), mW2p, b2p)

    agg = _scatter_agg(dst2d, msgs2)

    b = (b_ih + b_hh).reshape(1, 4 * H)
    Ws = [W_ih[i * H:(i + 1) * H] for i in range(4)]
    Us = [W_hh[i * H:(i + 1) * H] for i in range(4)]
    bs = [b[:, i * H:(i + 1) * H] for i in range(4)]
    h_new, c_new, out10 = _post(
        agg, puzzle, state_h, state_c, pW0, pb0.reshape(1, H), pW1,
        pb1.reshape(1, H), pW2, pb2.reshape(1, H), Ws, Us, bs, oW,
        ob.reshape(1, 10))
    return (h_new, c_new, out10.reshape(-1, 81, 10))


# speedup vs baseline: 9.3339x; 1.1739x over previous
import functools

import jax
import jax.numpy as jnp
from jax import lax
from jax.experimental import pallas as pl
from jax.experimental.pallas import tpu as pltpu
from jax.experimental.pallas import tpu_sc as plsc

N = 41472
E = 829440
H = 96
EMB = 16
DE = 16

_GW = 32
_GCHUNK = (2 * E) // _GW
_GSUB = 6480
_GNSUB = _GCHUNK // _GSUB


def _gather_cls(nodes, eflat):
    mesh = plsc.VectorSubcoreMesh(core_axis_name="c", subcore_axis_name="s")

    @functools.partial(
        pl.kernel,
        out_type=jax.ShapeDtypeStruct((2 * E,), jnp.int32),
        mesh=mesh,
        scratch_types=[
            pltpu.VMEM_SHARED((N,), jnp.int32),
            pltpu.VMEM((_GSUB,), jnp.int32),
            pltpu.VMEM((_GSUB,), jnp.int32),
            pltpu.VMEM((_GSUB,), jnp.int32),
            pltpu.VMEM((_GSUB,), jnp.int32),
            pltpu.SemaphoreType.DMA((2,)),
            pltpu.SemaphoreType.DMA,
        ],
    )
    def k(nodes_hbm, eflat_hbm, out_hbm, tbl, ibuf0, ibuf1, obuf0, obuf1,
          isem, gsem):
        ibufs = (ibuf0, ibuf1)
        obufs = (obuf0, obuf1)
        cid = lax.axis_index("c")
        sid = lax.axis_index("s")
        wid = sid * 2 + cid
        base = wid * _GCHUNK

        @pl.when(sid == 0)
        def _():
            pltpu.sync_copy(nodes_hbm, tbl)
        plsc.subcore_barrier()

        pltpu.async_copy(eflat_hbm.at[pl.ds(base, _GSUB)], ibufs[0],
                         isem.at[0])
        for sc in range(_GNSUB):
            b = sc % 2
            off = base + sc * _GSUB
            if sc + 1 < _GNSUB:
                pltpu.async_copy(
                    eflat_hbm.at[pl.ds(off + _GSUB, _GSUB)],
                    ibufs[1 - b], isem.at[1 - b])
            pltpu.make_async_copy(eflat_hbm.at[pl.ds(off, _GSUB)],
                                  ibufs[b], isem.at[b]).wait()
            pltpu.async_copy(tbl.at[ibufs[b]], obufs[b], gsem).wait()
            pltpu.sync_copy(obufs[b], out_hbm.at[pl.ds(off, _GSUB)])

    return k(nodes, eflat)


_SG = 24
_SROWS_PER_TILE = (E // 128) // 16
_SCHUNK_ROWS = 9
_SCHUNK = _SCHUNK_ROWS * 128
_SNCHUNK = _SROWS_PER_TILE // _SCHUNK_ROWS
_SZROWS = 162
_NPT = N // 16


def _scatter_agg(dst2d, msgs2):
    mesh = plsc.VectorSubcoreMesh(core_axis_name="c", subcore_axis_name="s")

    @functools.partial(
        pl.kernel,
        out_type=jax.ShapeDtypeStruct((4, N, _SG), jnp.float32),
        mesh=mesh,
        scratch_types=[
            pltpu.VMEM_SHARED((N, _SG), jnp.float32),
            pltpu.VMEM((_SZROWS, _SG), jnp.float32),
            pltpu.VMEM((2, _SCHUNK, _SG), jnp.float32),
            pltpu.VMEM((2, _SCHUNK_ROWS, 128), jnp.int32),
            pltpu.SemaphoreType.DMA((2,)),
            pltpu.SemaphoreType.DMA((2,)),
            pltpu.SemaphoreType.DMA,
        ],
        compiler_params=pltpu.CompilerParams(use_tc_tiling_on_sc=False),
    )
    def k(dst2d_hbm, msgs2_hbm, out_hbm, acc, zbuf, dbuf, ibuf, dsem, isem,
          ssem):
        cid = lax.axis_index("c")
        sid = lax.axis_index("s")

        zeros16 = jnp.zeros((16,), jnp.float32)

        def zrow(i, carry):
            zbuf[i, pl.ds(0, 16)] = zeros16
            zbuf[i, pl.ds(8, 16)] = zeros16
            return carry

        lax.fori_loop(0, _SZROWS, zrow, 0)

        for p in range(2):
            grp = cid * 2 + p
            col0 = grp * _SG
            for t in range(_NPT // _SZROWS):
                pltpu.sync_copy(
                    zbuf, acc.at[pl.ds(sid * _NPT + t * _SZROWS, _SZROWS)])
            plsc.subcore_barrier()

            def start_in(t, b):
                row0 = sid * _SROWS_PER_TILE + t * _SCHUNK_ROWS
                pltpu.async_copy(
                    msgs2_hbm.at[pl.ds(row0 * 128, _SCHUNK),
                                 pl.ds(col0, _SG)],
                    dbuf.at[b], dsem.at[b])
                pltpu.async_copy(dst2d_hbm.at[pl.ds(row0, _SCHUNK_ROWS)],
                                 ibuf.at[b], isem.at[b])

            def wait_in(t, b):
                row0 = sid * _SROWS_PER_TILE + t * _SCHUNK_ROWS
                pltpu.make_async_copy(
                    msgs2_hbm.at[pl.ds(row0 * 128, _SCHUNK),
                                 pl.ds(col0, _SG)],
                    dbuf.at[b], dsem.at[b]).wait()
                pltpu.make_async_copy(
                    dst2d_hbm.at[pl.ds(row0, _SCHUNK_ROWS)],
                    ibuf.at[b], isem.at[b]).wait()

            start_in(0, 0)

            def chunk2(t2, carry):
                for b in range(2):
                    t = t2 * 2 + b

                    @pl.when(t < _SNCHUNK)
                    def _():
                        @pl.when(t + 1 < _SNCHUNK)
                        def _():
                            start_in(t + 1, 1 - b)
                        wait_in(t, b)
                        descs = []
                        for j in range(_SCHUNK_ROWS):
                            descs.append(pltpu.async_copy(
                                dbuf.at[b, pl.ds(j * 128, 128)],
                                acc.at[ibuf.at[b, j]], ssem, add=True))
                        for d in descs:
                            d.wait()
                return carry

            lax.fori_loop(0, (_SNCHUNK + 1) // 2, chunk2, 0)
            plsc.subcore_barrier()

            pltpu.sync_copy(acc.at[pl.ds(sid * _NPT, _NPT)],
                            out_hbm.at[grp, pl.ds(sid * _NPT, _NPT)])

    return k(dst2d, msgs2)


_EB = 5120
_ENB = E // _EB


def _edge_body(cs_ref, cd_ref, eft_ref, emb_ref, mW0_ref, b0_ref, mW1_ref,
               b1_ref, mW2p_ref, b2p_ref, out_ref):
    f32 = jnp.float32
    dims11 = (((1,), (1,)), ((), ()))
    dims10 = (((1,), (0,)), ((), ()))
    dims01 = (((0,), (1,)), ((), ()))
    cs = cs_ref[0, 0, :]
    cd = cd_ref[0, 0, :]
    iota10 = lax.broadcasted_iota(jnp.int32, (1, 10), 1)
    oh_s = (cs[:, None] == iota10).astype(f32)
    oh_d = (cd[:, None] == iota10).astype(f32)
    emb = emb_ref[...]
    TA = lax.dot_general(emb, mW0_ref[:, 0:EMB], dims11,
                         preferred_element_type=f32)
    TB = lax.dot_general(emb, mW0_ref[:, EMB:2 * EMB], dims11,
                         preferred_element_type=f32)
    h0 = (lax.dot_general(oh_s, TA, dims10, preferred_element_type=f32)
          + lax.dot_general(oh_d, TB, dims10, preferred_element_type=f32)
          + lax.dot_general(eft_ref[...], mW0_ref[:, 2 * EMB:], dims01,
                            preferred_element_type=f32)
          + b0_ref[...])
    bf16 = jnp.bfloat16
    h1 = jnp.maximum(h0, 0.0).astype(bf16)
    h2 = jnp.maximum(
        lax.dot_general(h1, mW1_ref[...].astype(bf16), dims11,
                        preferred_element_type=f32)
        + b1_ref[...], 0.0).astype(bf16)
    out_ref[...] = lax.dot_general(h2, mW2p_ref[...].astype(bf16), dims11,
                                   preferred_element_type=f32) + b2p_ref[...]


def _edge_mlp(cs3, cd3, eft, emb_s, mW0, b0r, mW1, b1r, mW2p, b2pr):
    full = lambda shape: pl.BlockSpec(shape, lambda i, _s=shape: tuple(0 for _ in _s))
    return pl.pallas_call(
        _edge_body,
        grid=(_ENB,),
        in_specs=[
            pl.BlockSpec((1, 1, _EB), lambda i: (i, 0, 0)),
            pl.BlockSpec((1, 1, _EB), lambda i: (i, 0, 0)),
            pl.BlockSpec((DE, _EB), lambda i: (0, i)),
            full((10, EMB)),
            full((H, 2 * EMB + DE)),
            full((1, H)),
            full((H, H)),
            full((1, H)),
            full((128, H)),
            full((1, 128)),
        ],
        out_specs=pl.BlockSpec((_EB, 128), lambda i: (i, 0)),
        out_shape=jax.ShapeDtypeStruct((E, 128), jnp.float32),
        compiler_params=pltpu.CompilerParams(
            dimension_semantics=("arbitrary",)),
    )(cs3, cd3, eft, emb_s, mW0, b0r, mW1, b1r, mW2p, b2pr)


_RB = 1296
_RNB = N // _RB


def _post_body(agg_ref, pz_ref, sh_ref, sc_ref, pW0_ref, pb0_ref, pW1_ref,
               pb1_ref, pW2_ref, pb2_ref, Wi_ref, Wf_ref, Wg_ref, Wo_ref,
               Ui_ref, Uf_ref, Ug_ref, Uo_ref, bi_ref, bf_ref, bg_ref,
               bo_ref, oW_ref, ob_ref, h_ref, c_ref, o_ref):
    f32 = jnp.float32
    dims11 = (((1,), (1,)), ((), ()))

    def dot(x, w):
        return lax.dot_general(x, w, dims11, preferred_element_type=f32)

    g0 = (dot(agg_ref[0], pW0_ref[:, 0:_SG])
          + dot(agg_ref[1], pW0_ref[:, _SG:2 * _SG])
          + dot(agg_ref[2], pW0_ref[:, 2 * _SG:3 * _SG])
          + dot(agg_ref[3], pW0_ref[:, 3 * _SG:4 * _SG])
          + dot(pz_ref[...], pW0_ref[:, 96:112]) + pb0_ref[...])
    h = jnp.maximum(g0, 0.0)
    h = jnp.maximum(dot(h, pW1_ref[...]) + pb1_ref[...], 0.0)
    hp = dot(h, pW2_ref[...]) + pb2_ref[...]
    sh = sh_ref[...]
    ii = jax.nn.sigmoid(dot(hp, Wi_ref[...]) + dot(sh, Ui_ref[...]) + bi_ref[...])
    ff = jax.nn.sigmoid(dot(hp, Wf_ref[...]) + dot(sh, Uf_ref[...]) + bf_ref[...])
    gg = jnp.tanh(dot(hp, Wg_ref[...]) + dot(sh, Ug_ref[...]) + bg_ref[...])
    oo = jax.nn.sigmoid(dot(hp, Wo_ref[...]) + dot(sh, Uo_ref[...]) + bo_ref[...])
    cn = ff * sc_ref[...] + ii * gg
    hn = oo * jnp.tanh(cn)
    h_ref[...] = hn
    c_ref[...] = cn
    o_ref[...] = dot(hn, oW_ref[...]) + ob_ref[...]


def _post(agg, puzzle, sh, sc, pW0, pb0r, pW1, pb1r, pW2, pb2r, Ws, Us, bs,
          oW, obr):
    full = lambda shape: pl.BlockSpec(shape, lambda i, _s=shape: tuple(0 for _ in _s))
    return pl.pallas_call(
        _post_body,
        grid=(_RNB,),
        in_specs=[
            pl.BlockSpec((4, _RB, _SG), lambda i: (0, i, 0)),
            pl.BlockSpec((_RB, EMB), lambda i: (i, 0)),
            pl.BlockSpec((_RB, H), lambda i: (i, 0)),
            pl.BlockSpec((_RB, H), lambda i: (i, 0)),
            full((H, H + EMB)),
            full((1, H)),
            full((H, H)),
            full((1, H)),
            full((H, H)),
            full((1, H)),
            *[full((H, H)) for _ in range(8)],
            *[full((1, H)) for _ in range(4)],
            full((10, H)),
            full((1, 10)),
        ],
        out_specs=[
            pl.BlockSpec((_RB, H), lambda i: (i, 0)),
            pl.BlockSpec((_RB, H), lambda i: (i, 0)),
            pl.BlockSpec((_RB, 10), lambda i: (i, 0)),
        ],
        out_shape=[
            jax.ShapeDtypeStruct((N, H), jnp.float32),
            jax.ShapeDtypeStruct((N, H), jnp.float32),
            jax.ShapeDtypeStruct((N, 10), jnp.float32),
        ],
        compiler_params=pltpu.CompilerParams(
            dimension_semantics=("arbitrary",)),
    )(agg, puzzle, sh, sc, pW0, pb0r, pW1, pb1r, pW2, pb2r, *Ws, *Us, *bs,
      oW, obr)


def kernel(puzzle, nodes, edges, edge_features, state_h, state_c, first, emb,
           mW0, mb0, mW1, mb1, mW2, mb2, pW0, pb0, pW1, pb1, pW2, pb2, W_ih,
           W_hh, b_ih, b_hh, oW, ob):
    f32 = jnp.float32
    nodes = nodes.astype(jnp.int32)
    eflat = edges.astype(jnp.int32).reshape(2 * E)
    dst2d = edges[1].astype(jnp.int32).reshape(E // 128, 128)

    cls = _gather_cls(nodes, eflat)
    cs3 = cls[:E].reshape(_ENB, 1, _EB)
    cd3 = cls[E:].reshape(_ENB, 1, _EB)

    emb_s = emb * jnp.asarray(first, f32)
    mW2p = jnp.concatenate([mW2, jnp.zeros((128 - H, H), f32)], axis=0)
    b2p = jnp.concatenate([mb2, jnp.zeros((128 - H,), f32)]).reshape(1, 128)
    msgs2 = _edge_mlp(
        cs3, cd3, edge_features.T, emb_s, mW0, mb0.reshape(1, H), mW1,
        mb1.reshape(1, H), mW2p, b2p)

    agg = _scatter_agg(dst2d, msgs2)

    b = (b_ih + b_hh).reshape(1, 4 * H)
    Ws = [W_ih[i * H:(i + 1) * H] for i in range(4)]
    Us = [W_hh[i * H:(i + 1) * H] for i in range(4)]
    bs = [b[:, i * H:(i + 1) * H] for i in range(4)]
    h_new, c_new, out10 = _post(
        agg, puzzle, state_h, state_c, pW0, pb0.reshape(1, H), pW1,
        pb1.reshape(1, H), pW2, pb2.reshape(1, H), Ws, Us, bs, oW,
        ob.reshape(1, 10))
    return (h_new, c_new, out10.reshape(-1, 81, 10))


# speedup vs baseline: 11.1047x; 1.1897x over previous
import functools

import jax
import jax.numpy as jnp
from jax import lax
from jax.experimental import pallas as pl
from jax.experimental.pallas import tpu as pltpu
from jax.experimental.pallas import tpu_sc as plsc

N = 41472
E = 829440
H = 96
EMB = 16
DE = 16

_GW = 32
_GCHUNK = (2 * E) // _GW
_GSUB = 6480
_GNSUB = _GCHUNK // _GSUB


def _gather_cls(nodes, eflat):
    mesh = plsc.VectorSubcoreMesh(core_axis_name="c", subcore_axis_name="s")

    @functools.partial(
        pl.kernel,
        out_type=jax.ShapeDtypeStruct((2 * E,), jnp.int32),
        mesh=mesh,
        scratch_types=[
            pltpu.VMEM_SHARED((N,), jnp.int32),
            pltpu.VMEM((_GSUB,), jnp.int32),
            pltpu.VMEM((_GSUB,), jnp.int32),
            pltpu.VMEM((_GSUB,), jnp.int32),
            pltpu.VMEM((_GSUB,), jnp.int32),
            pltpu.SemaphoreType.DMA((2,)),
            pltpu.SemaphoreType.DMA,
        ],
    )
    def k(nodes_hbm, eflat_hbm, out_hbm, tbl, ibuf0, ibuf1, obuf0, obuf1,
          isem, gsem):
        ibufs = (ibuf0, ibuf1)
        obufs = (obuf0, obuf1)
        cid = lax.axis_index("c")
        sid = lax.axis_index("s")
        wid = sid * 2 + cid
        base = wid * _GCHUNK

        @pl.when(sid == 0)
        def _():
            pltpu.sync_copy(nodes_hbm, tbl)
        plsc.subcore_barrier()

        pltpu.async_copy(eflat_hbm.at[pl.ds(base, _GSUB)], ibufs[0],
                         isem.at[0])
        for sc in range(_GNSUB):
            b = sc % 2
            off = base + sc * _GSUB
            if sc + 1 < _GNSUB:
                pltpu.async_copy(
                    eflat_hbm.at[pl.ds(off + _GSUB, _GSUB)],
                    ibufs[1 - b], isem.at[1 - b])
            pltpu.make_async_copy(eflat_hbm.at[pl.ds(off, _GSUB)],
                                  ibufs[b], isem.at[b]).wait()
            pltpu.async_copy(tbl.at[ibufs[b]], obufs[b], gsem).wait()
            pltpu.sync_copy(obufs[b], out_hbm.at[pl.ds(off, _GSUB)])

    return k(nodes, eflat)


_SG = 24
_SROWS_PER_TILE = (E // 128) // 16
_SCHUNK_ROWS = 9
_SCHUNK = _SCHUNK_ROWS * 128
_SNCHUNK = _SROWS_PER_TILE // _SCHUNK_ROWS
_SZROWS = 162
_NPT = N // 16


def _scatter_agg(dst2d, msgs2):
    mesh = plsc.VectorSubcoreMesh(core_axis_name="c", subcore_axis_name="s")

    @functools.partial(
        pl.kernel,
        out_type=jax.ShapeDtypeStruct((N, 128), jnp.float32),
        mesh=mesh,
        scratch_types=[
            pltpu.VMEM_SHARED((N, _SG), jnp.float32),
            pltpu.VMEM((_SZROWS, _SG), jnp.float32),
            pltpu.VMEM((2, _SCHUNK, _SG), jnp.float32),
            pltpu.VMEM((2, _SCHUNK_ROWS, 128), jnp.int32),
            pltpu.SemaphoreType.DMA((2,)),
            pltpu.SemaphoreType.DMA((2,)),
            pltpu.SemaphoreType.DMA,
        ],
        compiler_params=pltpu.CompilerParams(use_tc_tiling_on_sc=False),
    )
    def k(dst2d_hbm, msgs2_hbm, out_hbm, acc, zbuf, dbuf, ibuf, dsem, isem,
          ssem):
        cid = lax.axis_index("c")
        sid = lax.axis_index("s")

        zeros16 = jnp.zeros((16,), jnp.float32)

        def zrow(i, carry):
            zbuf[i, pl.ds(0, 16)] = zeros16
            zbuf[i, pl.ds(8, 16)] = zeros16
            return carry

        lax.fori_loop(0, _SZROWS, zrow, 0)

        for p in range(2):
            grp = cid * 2 + p
            col0 = grp * _SG
            for t in range(_NPT // _SZROWS):
                pltpu.sync_copy(
                    zbuf, acc.at[pl.ds(sid * _NPT + t * _SZROWS, _SZROWS)])
            plsc.subcore_barrier()

            def start_in(t, b):
                row0 = sid * _SROWS_PER_TILE + t * _SCHUNK_ROWS
                pltpu.async_copy(
                    msgs2_hbm.at[pl.ds(row0 * 128, _SCHUNK),
                                 pl.ds(col0, _SG)],
                    dbuf.at[b], dsem.at[b])
                pltpu.async_copy(dst2d_hbm.at[pl.ds(row0, _SCHUNK_ROWS)],
                                 ibuf.at[b], isem.at[b])

            def wait_in(t, b):
                row0 = sid * _SROWS_PER_TILE + t * _SCHUNK_ROWS
                pltpu.make_async_copy(
                    msgs2_hbm.at[pl.ds(row0 * 128, _SCHUNK),
                                 pl.ds(col0, _SG)],
                    dbuf.at[b], dsem.at[b]).wait()
                pltpu.make_async_copy(
                    dst2d_hbm.at[pl.ds(row0, _SCHUNK_ROWS)],
                    ibuf.at[b], isem.at[b]).wait()

            start_in(0, 0)

            def chunk2(t2, carry):
                for b in range(2):
                    t = t2 * 2 + b

                    @pl.when(t < _SNCHUNK)
                    def _():
                        @pl.when(t + 1 < _SNCHUNK)
                        def _():
                            start_in(t + 1, 1 - b)
                        wait_in(t, b)
                        descs = []
                        for j in range(_SCHUNK_ROWS):
                            descs.append(pltpu.async_copy(
                                dbuf.at[b, pl.ds(j * 128, 128)],
                                acc.at[ibuf.at[b, j]], ssem, add=True))
                        for d in descs:
                            d.wait()
                return carry

            lax.fori_loop(0, (_SNCHUNK + 1) // 2, chunk2, 0)
            plsc.subcore_barrier()

            pltpu.sync_copy(acc.at[pl.ds(sid * _NPT, _NPT)],
                            out_hbm.at[pl.ds(sid * _NPT, _NPT),
                                       pl.ds(col0, _SG)])

    return k(dst2d, msgs2)


_EB = 5120
_ENB = E // _EB


def _edge_body(cs_ref, cd_ref, eft_ref, emb_ref, mW0_ref, b0_ref, mW1_ref,
               b1_ref, mW2p_ref, b2p_ref, out_ref):
    f32 = jnp.float32
    dims11 = (((1,), (1,)), ((), ()))
    dims10 = (((1,), (0,)), ((), ()))
    dims01 = (((0,), (1,)), ((), ()))
    cs = cs_ref[0, 0, :]
    cd = cd_ref[0, 0, :]
    iota10 = lax.broadcasted_iota(jnp.int32, (1, 10), 1)
    oh_s = (cs[:, None] == iota10).astype(f32)
    oh_d = (cd[:, None] == iota10).astype(f32)
    emb = emb_ref[...]
    TA = lax.dot_general(emb, mW0_ref[:, 0:EMB], dims11,
                         preferred_element_type=f32)
    TB = lax.dot_general(emb, mW0_ref[:, EMB:2 * EMB], dims11,
                         preferred_element_type=f32)
    h0 = (lax.dot_general(oh_s, TA, dims10, preferred_element_type=f32)
          + lax.dot_general(oh_d, TB, dims10, preferred_element_type=f32)
          + lax.dot_general(eft_ref[...], mW0_ref[:, 2 * EMB:], dims01,
                            preferred_element_type=f32)
          + b0_ref[...])
    bf16 = jnp.bfloat16
    h1 = jnp.maximum(h0, 0.0).astype(bf16)
    h2 = jnp.maximum(
        lax.dot_general(h1, mW1_ref[...].astype(bf16), dims11,
                        preferred_element_type=f32)
        + b1_ref[...], 0.0).astype(bf16)
    out_ref[...] = lax.dot_general(h2, mW2p_ref[...].astype(bf16), dims11,
                                   preferred_element_type=f32) + b2p_ref[...]


def _edge_mlp(cs3, cd3, eft, emb_s, mW0, b0r, mW1, b1r, mW2p, b2pr):
    full = lambda shape: pl.BlockSpec(shape, lambda i, _s=shape: tuple(0 for _ in _s))
    return pl.pallas_call(
        _edge_body,
        grid=(_ENB,),
        in_specs=[
            pl.BlockSpec((1, 1, _EB), lambda i: (i, 0, 0)),
            pl.BlockSpec((1, 1, _EB), lambda i: (i, 0, 0)),
            pl.BlockSpec((DE, _EB), lambda i: (0, i)),
            full((10, EMB)),
            full((H, 2 * EMB + DE)),
            full((1, H)),
            full((H, H)),
            full((1, H)),
            full((128, H)),
            full((1, 128)),
        ],
        out_specs=pl.BlockSpec((_EB, 128), lambda i: (i, 0)),
        out_shape=jax.ShapeDtypeStruct((E, 128), jnp.float32),
        compiler_params=pltpu.CompilerParams(
            dimension_semantics=("arbitrary",)),
    )(cs3, cd3, eft, emb_s, mW0, b0r, mW1, b1r, mW2p, b2pr)


_RB = 2304
_RNB = N // _RB


def _post_body(agg_ref, pzt_ref, sht_ref, sct_ref, pW0_ref, pb0_ref, pW1_ref,
               pb1_ref, pW2_ref, pb2_ref, Wi_ref, Wf_ref, Wg_ref, Wo_ref,
               Ui_ref, Uf_ref, Ug_ref, Uo_ref, bi_ref, bf_ref, bg_ref,
               bo_ref, oW_ref, ob_ref, h_ref, c_ref, o_ref):
    f32 = jnp.float32
    dims11 = (((1,), (1,)), ((), ()))
    dims10 = (((1,), (0,)), ((), ()))

    def dott(w, x):
        return lax.dot_general(w, x, dims10, preferred_element_type=f32)

    agg96 = agg_ref[:, 0:H]
    g0 = (lax.dot_general(pW0_ref[:, 0:H], agg96, dims11,
                          preferred_element_type=f32)
          + dott(pW0_ref[:, H:H + EMB], pzt_ref[...]) + pb0_ref[...])
    h = jnp.maximum(g0, 0.0)
    h = jnp.maximum(dott(pW1_ref[...], h) + pb1_ref[...], 0.0)
    hp = dott(pW2_ref[...], h) + pb2_ref[...]
    sh = sht_ref[...]
    ii = jax.nn.sigmoid(dott(Wi_ref[...], hp) + dott(Ui_ref[...], sh) + bi_ref[...])
    ff = jax.nn.sigmoid(dott(Wf_ref[...], hp) + dott(Uf_ref[...], sh) + bf_ref[...])
    gg = jnp.tanh(dott(Wg_ref[...], hp) + dott(Ug_ref[...], sh) + bg_ref[...])
    oo = jax.nn.sigmoid(dott(Wo_ref[...], hp) + dott(Uo_ref[...], sh) + bo_ref[...])
    cn = ff * sct_ref[...] + ii * gg
    hn = oo * jnp.tanh(cn)
    h_ref[...] = hn
    c_ref[...] = cn
    o_ref[...] = dott(oW_ref[...], hn) + ob_ref[...]


def _post(agg, puzzlet, sht, sct, pW0, pb0c, pW1, pb1c, pW2, pb2c, Ws, Us,
          bs, oW, obc):
    full = lambda shape: pl.BlockSpec(shape, lambda i, _s=shape: tuple(0 for _ in _s))
    return pl.pallas_call(
        _post_body,
        grid=(_RNB,),
        in_specs=[
            pl.BlockSpec((_RB, 128), lambda i: (i, 0)),
            pl.BlockSpec((EMB, _RB), lambda i: (0, i)),
            pl.BlockSpec((H, _RB), lambda i: (0, i)),
            pl.BlockSpec((H, _RB), lambda i: (0, i)),
            full((H, H + EMB)),
            full((H, 1)),
            full((H, H)),
            full((H, 1)),
            full((H, H)),
            full((H, 1)),
            *[full((H, H)) for _ in range(8)],
            *[full((H, 1)) for _ in range(4)],
            full((10, H)),
            full((10, 1)),
        ],
        out_specs=[
            pl.BlockSpec((H, _RB), lambda i: (0, i)),
            pl.BlockSpec((H, _RB), lambda i: (0, i)),
            pl.BlockSpec((10, _RB), lambda i: (0, i)),
        ],
        out_shape=[
            jax.ShapeDtypeStruct((H, N), jnp.float32),
            jax.ShapeDtypeStruct((H, N), jnp.float32),
            jax.ShapeDtypeStruct((10, N), jnp.float32),
        ],
        compiler_params=pltpu.CompilerParams(
            dimension_semantics=("arbitrary",)),
    )(agg, puzzlet, sht, sct, pW0, pb0c, pW1, pb1c, pW2, pb2c, *Ws, *Us, *bs,
      oW, obc)


def kernel(puzzle, nodes, edges, edge_features, state_h, state_c, first, emb,
           mW0, mb0, mW1, mb1, mW2, mb2, pW0, pb0, pW1, pb1, pW2, pb2, W_ih,
           W_hh, b_ih, b_hh, oW, ob):
    f32 = jnp.float32
    nodes = nodes.astype(jnp.int32)
    eflat = edges.astype(jnp.int32).reshape(2 * E)
    dst2d = edges[1].astype(jnp.int32).reshape(E // 128, 128)

    cls = _gather_cls(nodes, eflat)
    cs3 = cls[:E].reshape(_ENB, 1, _EB)
    cd3 = cls[E:].reshape(_ENB, 1, _EB)

    emb_s = emb * jnp.asarray(first, f32)
    mW2p = jnp.concatenate([mW2, jnp.zeros((128 - H, H), f32)], axis=0)
    b2p = jnp.concatenate([mb2, jnp.zeros((128 - H,), f32)]).reshape(1, 128)
    msgs2 = _edge_mlp(
        cs3, cd3, edge_features.T, emb_s, mW0, mb0.reshape(1, H), mW1,
        mb1.reshape(1, H), mW2p, b2p)

    agg = _scatter_agg(dst2d, msgs2)

    b = b_ih + b_hh
    Ws = [W_ih[i * H:(i + 1) * H] for i in range(4)]
    Us = [W_hh[i * H:(i + 1) * H] for i in range(4)]
    bs = [b[i * H:(i + 1) * H].reshape(H, 1) for i in range(4)]
    ht, ct, outt = _post(
        agg, puzzle.T, state_h.T, state_c.T, pW0, pb0.reshape(H, 1), pW1,
        pb1.reshape(H, 1), pW2, pb2.reshape(H, 1), Ws, Us, bs, oW,
        ob.reshape(10, 1))
    return (ht.T, ct.T, outt.T.reshape(-1, 81, 10))


# speedup vs baseline: 11.5498x; 1.0401x over previous
import functools

import jax
import jax.numpy as jnp
from jax import lax
from jax.experimental import pallas as pl
from jax.experimental.pallas import tpu as pltpu
from jax.experimental.pallas import tpu_sc as plsc

N = 41472
E = 829440
H = 96
EMB = 16
DE = 16

_GW = 32
_GCHUNK = (2 * E) // _GW
_GSUB = 6480
_GNSUB = _GCHUNK // _GSUB


def _gather_cls(nodes, eflat):
    mesh = plsc.VectorSubcoreMesh(core_axis_name="c", subcore_axis_name="s")

    @functools.partial(
        pl.kernel,
        out_type=jax.ShapeDtypeStruct((2 * E,), jnp.int32),
        mesh=mesh,
        scratch_types=[
            pltpu.VMEM_SHARED((N,), jnp.int32),
            pltpu.VMEM((_GSUB,), jnp.int32),
            pltpu.VMEM((_GSUB,), jnp.int32),
            pltpu.VMEM((_GSUB,), jnp.int32),
            pltpu.VMEM((_GSUB,), jnp.int32),
            pltpu.SemaphoreType.DMA((2,)),
            pltpu.SemaphoreType.DMA,
        ],
    )
    def k(nodes_hbm, eflat_hbm, out_hbm, tbl, ibuf0, ibuf1, obuf0, obuf1,
          isem, gsem):
        ibufs = (ibuf0, ibuf1)
        obufs = (obuf0, obuf1)
        cid = lax.axis_index("c")
        sid = lax.axis_index("s")
        wid = sid * 2 + cid
        base = wid * _GCHUNK

        @pl.when(sid == 0)
        def _():
            pltpu.sync_copy(nodes_hbm, tbl)
        plsc.subcore_barrier()

        pltpu.async_copy(eflat_hbm.at[pl.ds(base, _GSUB)], ibufs[0],
                         isem.at[0])
        for sc in range(_GNSUB):
            b = sc % 2
            off = base + sc * _GSUB
            if sc + 1 < _GNSUB:
                pltpu.async_copy(
                    eflat_hbm.at[pl.ds(off + _GSUB, _GSUB)],
                    ibufs[1 - b], isem.at[1 - b])
            pltpu.make_async_copy(eflat_hbm.at[pl.ds(off, _GSUB)],
                                  ibufs[b], isem.at[b]).wait()
            pltpu.async_copy(tbl.at[ibufs[b]], obufs[b], gsem).wait()
            pltpu.sync_copy(obufs[b], out_hbm.at[pl.ds(off, _GSUB)])

    return k(nodes, eflat)


_SG = 24
_SROWS_PER_TILE = (E // 128) // 16
_SCHUNK_ROWS = 9
_SCHUNK = _SCHUNK_ROWS * 128
_SNCHUNK = _SROWS_PER_TILE // _SCHUNK_ROWS
_SZROWS = 162
_NPT = N // 16


def _scatter_agg(dst2d, msgs2):
    mesh = plsc.VectorSubcoreMesh(core_axis_name="c", subcore_axis_name="s")

    @functools.partial(
        pl.kernel,
        out_type=jax.ShapeDtypeStruct((N, 128), jnp.float32),
        mesh=mesh,
        scratch_types=[
            pltpu.VMEM_SHARED((N, _SG), jnp.float32),
            pltpu.VMEM((_SZROWS, _SG), jnp.float32),
            pltpu.VMEM((2, _SCHUNK, _SG), jnp.float32),
            pltpu.VMEM((2, _SCHUNK_ROWS, 128), jnp.int32),
            pltpu.SemaphoreType.DMA((2,)),
            pltpu.SemaphoreType.DMA((2,)),
            pltpu.SemaphoreType.DMA,
        ],
        compiler_params=pltpu.CompilerParams(use_tc_tiling_on_sc=False),
    )
    def k(dst2d_hbm, msgs2_hbm, out_hbm, acc, zbuf, dbuf, ibuf, dsem, isem,
          ssem):
        cid = lax.axis_index("c")
        sid = lax.axis_index("s")

        zeros16 = jnp.zeros((16,), jnp.float32)

        def zrow(i, carry):
            zbuf[i, pl.ds(0, 16)] = zeros16
            zbuf[i, pl.ds(8, 16)] = zeros16
            return carry

        lax.fori_loop(0, _SZROWS, zrow, 0)

        for p in range(2):
            grp = cid * 2 + p
            col0 = grp * _SG
            for t in range(_NPT // _SZROWS):
                pltpu.sync_copy(
                    zbuf, acc.at[pl.ds(sid * _NPT + t * _SZROWS, _SZROWS)])
            plsc.subcore_barrier()

            def start_in(t, b):
                row0 = sid * _SROWS_PER_TILE + t * _SCHUNK_ROWS
                pltpu.async_copy(
                    msgs2_hbm.at[pl.ds(row0 * 128, _SCHUNK),
                                 pl.ds(col0, _SG)],
                    dbuf.at[b], dsem.at[b])
                pltpu.async_copy(dst2d_hbm.at[pl.ds(row0, _SCHUNK_ROWS)],
                                 ibuf.at[b], isem.at[b])

            def wait_in(t, b):
                row0 = sid * _SROWS_PER_TILE + t * _SCHUNK_ROWS
                pltpu.make_async_copy(
                    msgs2_hbm.at[pl.ds(row0 * 128, _SCHUNK),
                                 pl.ds(col0, _SG)],
                    dbuf.at[b], dsem.at[b]).wait()
                pltpu.make_async_copy(
                    dst2d_hbm.at[pl.ds(row0, _SCHUNK_ROWS)],
                    ibuf.at[b], isem.at[b]).wait()

            start_in(0, 0)

            def chunk2(t2, carry):
                for b in range(2):
                    t = t2 * 2 + b

                    @pl.when(t < _SNCHUNK)
                    def _():
                        @pl.when(t + 1 < _SNCHUNK)
                        def _():
                            start_in(t + 1, 1 - b)
                        wait_in(t, b)
                        descs = []
                        for j in range(_SCHUNK_ROWS):
                            descs.append(pltpu.async_copy(
                                dbuf.at[b, pl.ds(j * 128, 128)],
                                acc.at[ibuf.at[b, j]], ssem, add=True))
                        for d in descs:
                            d.wait()
                return carry

            lax.fori_loop(0, (_SNCHUNK + 1) // 2, chunk2, 0)
            plsc.subcore_barrier()

            pltpu.sync_copy(acc.at[pl.ds(sid * _NPT, _NPT)],
                            out_hbm.at[pl.ds(sid * _NPT, _NPT),
                                       pl.ds(col0, _SG)])

    return k(dst2d, msgs2)


_EB = 5120
_ENB = E // _EB


def _edge_body(cs_ref, cd_ref, eft_ref, emb_ref, mW0_ref, b0_ref, mW1_ref,
               b1_ref, mW2p_ref, b2p_ref, out_ref):
    f32 = jnp.float32
    dims11 = (((1,), (1,)), ((), ()))
    dims00 = (((0,), (0,)), ((), ()))
    dims01 = (((0,), (1,)), ((), ()))
    cs = cs_ref[0]
    cd = cd_ref[0]
    iota10c = lax.broadcasted_iota(jnp.int32, (10, 1), 0)
    oh_st = (cs == iota10c).astype(f32)
    oh_dt = (cd == iota10c).astype(f32)
    emb = emb_ref[...]
    TA = lax.dot_general(emb, mW0_ref[:, 0:EMB], dims11,
                         preferred_element_type=f32)
    TB = lax.dot_general(emb, mW0_ref[:, EMB:2 * EMB], dims11,
                         preferred_element_type=f32)
    h0 = (lax.dot_general(oh_st, TA, dims00, preferred_element_type=f32)
          + lax.dot_general(oh_dt, TB, dims00, preferred_element_type=f32)
          + lax.dot_general(eft_ref[...], mW0_ref[:, 2 * EMB:], dims01,
                            preferred_element_type=f32)
          + b0_ref[...])
    bf16 = jnp.bfloat16
    h1 = jnp.maximum(h0, 0.0).astype(bf16)
    h2 = jnp.maximum(
        lax.dot_general(h1, mW1_ref[...].astype(bf16), dims11,
                        preferred_element_type=f32)
        + b1_ref[...], 0.0).astype(bf16)
    out_ref[...] = lax.dot_general(h2, mW2p_ref[...].astype(bf16), dims11,
                                   preferred_element_type=f32) + b2p_ref[...]


def _edge_mlp(cs3, cd3, eft, emb_s, mW0, b0r, mW1, b1r, mW2p, b2pr):
    full = lambda shape: pl.BlockSpec(shape, lambda i, _s=shape: tuple(0 for _ in _s))
    return pl.pallas_call(
        _edge_body,
        grid=(_ENB,),
        in_specs=[
            pl.BlockSpec((1, 1, _EB), lambda i: (i, 0, 0)),
            pl.BlockSpec((1, 1, _EB), lambda i: (i, 0, 0)),
            pl.BlockSpec((DE, _EB), lambda i: (0, i)),
            full((10, EMB)),
            full((H, 2 * EMB + DE)),
            full((1, H)),
            full((H, H)),
            full((1, H)),
            full((128, H)),
            full((1, 128)),
        ],
        out_specs=pl.BlockSpec((_EB, 128), lambda i: (i, 0)),
        out_shape=jax.ShapeDtypeStruct((E, 128), jnp.float32),
        compiler_params=pltpu.CompilerParams(
            dimension_semantics=("arbitrary",)),
    )(cs3, cd3, eft, emb_s, mW0, b0r, mW1, b1r, mW2p, b2pr)


_RB = 2304
_RNB = N // _RB


def _post_body(agg_ref, pzt_ref, sht_ref, sct_ref, pW0_ref, pb0_ref, pW1_ref,
               pb1_ref, pW2_ref, pb2_ref, Wi_ref, Wf_ref, Wg_ref, Wo_ref,
               Ui_ref, Uf_ref, Ug_ref, Uo_ref, bi_ref, bf_ref, bg_ref,
               bo_ref, oW_ref, ob_ref, h_ref, c_ref, o_ref):
    f32 = jnp.float32
    dims11 = (((1,), (1,)), ((), ()))
    dims10 = (((1,), (0,)), ((), ()))

    def dott(w, x):
        return lax.dot_general(w, x, dims10, preferred_element_type=f32)

    agg96 = agg_ref[:, 0:H]
    g0 = (lax.dot_general(pW0_ref[:, 0:H], agg96, dims11,
                          preferred_element_type=f32)
          + dott(pW0_ref[:, H:H + EMB], pzt_ref[...]) + pb0_ref[...])
    h = jnp.maximum(g0, 0.0)
    h = jnp.maximum(dott(pW1_ref[...], h) + pb1_ref[...], 0.0)
    hp = dott(pW2_ref[...], h) + pb2_ref[...]
    sh = sht_ref[...]
    ii = jax.nn.sigmoid(dott(Wi_ref[...], hp) + dott(Ui_ref[...], sh) + bi_ref[...])
    ff = jax.nn.sigmoid(dott(Wf_ref[...], hp) + dott(Uf_ref[...], sh) + bf_ref[...])
    gg = jnp.tanh(dott(Wg_ref[...], hp) + dott(Ug_ref[...], sh) + bg_ref[...])
    oo = jax.nn.sigmoid(dott(Wo_ref[...], hp) + dott(Uo_ref[...], sh) + bo_ref[...])
    cn = ff * sct_ref[...] + ii * gg
    hn = oo * jnp.tanh(cn)
    h_ref[...] = hn
    c_ref[...] = cn
    o_ref[...] = dott(oW_ref[...], hn) + ob_ref[...]


def _post(agg, puzzlet, sht, sct, pW0, pb0c, pW1, pb1c, pW2, pb2c, Ws, Us,
          bs, oW, obc):
    full = lambda shape: pl.BlockSpec(shape, lambda i, _s=shape: tuple(0 for _ in _s))
    return pl.pallas_call(
        _post_body,
        grid=(_RNB,),
        in_specs=[
            pl.BlockSpec((_RB, 128), lambda i: (i, 0)),
            pl.BlockSpec((EMB, _RB), lambda i: (0, i)),
            pl.BlockSpec((H, _RB), lambda i: (0, i)),
            pl.BlockSpec((H, _RB), lambda i: (0, i)),
            full((H, H + EMB)),
            full((H, 1)),
            full((H, H)),
            full((H, 1)),
            full((H, H)),
            full((H, 1)),
            *[full((H, H)) for _ in range(8)],
            *[full((H, 1)) for _ in range(4)],
            full((10, H)),
            full((10, 1)),
        ],
        out_specs=[
            pl.BlockSpec((H, _RB), lambda i: (0, i)),
            pl.BlockSpec((H, _RB), lambda i: (0, i)),
            pl.BlockSpec((10, _RB), lambda i: (0, i)),
        ],
        out_shape=[
            jax.ShapeDtypeStruct((H, N), jnp.float32),
            jax.ShapeDtypeStruct((H, N), jnp.float32),
            jax.ShapeDtypeStruct((10, N), jnp.float32),
        ],
        compiler_params=pltpu.CompilerParams(
            dimension_semantics=("arbitrary",)),
    )(agg, puzzlet, sht, sct, pW0, pb0c, pW1, pb1c, pW2, pb2c, *Ws, *Us, *bs,
      oW, obc)


def kernel(puzzle, nodes, edges, edge_features, state_h, state_c, first, emb,
           mW0, mb0, mW1, mb1, mW2, mb2, pW0, pb0, pW1, pb1, pW2, pb2, W_ih,
           W_hh, b_ih, b_hh, oW, ob):
    f32 = jnp.float32
    nodes = nodes.astype(jnp.int32)
    eflat = edges.astype(jnp.int32).reshape(2 * E)
    dst2d = edges[1].astype(jnp.int32).reshape(E // 128, 128)

    cls = _gather_cls(nodes, eflat)
    cs3 = cls[:E].reshape(_ENB, 1, _EB)
    cd3 = cls[E:].reshape(_ENB, 1, _EB)

    emb_s = emb * jnp.asarray(first, f32)
    mW2p = jnp.concatenate([mW2, jnp.zeros((128 - H, H), f32)], axis=0)
    b2p = jnp.concatenate([mb2, jnp.zeros((128 - H,), f32)]).reshape(1, 128)
    msgs2 = _edge_mlp(
        cs3, cd3, edge_features.T, emb_s, mW0, mb0.reshape(1, H), mW1,
        mb1.reshape(1, H), mW2p, b2p)

    agg = _scatter_agg(dst2d, msgs2)

    b = b_ih + b_hh
    Ws = [W_ih[i * H:(i + 1) * H] for i in range(4)]
    Us = [W_hh[i * H:(i + 1) * H] for i in range(4)]
    bs = [b[i * H:(i + 1) * H].reshape(H, 1) for i in range(4)]
    ht, ct, outt = _post(
        agg, puzzle.T, state_h.T, state_c.T, pW0, pb0.reshape(H, 1), pW1,
        pb1.reshape(H, 1), pW2, pb2.reshape(H, 1), Ws, Us, bs, oW,
        ob.reshape(10, 1))
    return (ht.T, ct.T, outt.T.reshape(-1, 81, 10))


# speedup vs baseline: 12.6387x; 1.0943x over previous
import functools

import jax
import jax.numpy as jnp
from jax import lax
from jax.experimental import pallas as pl
from jax.experimental.pallas import tpu as pltpu
from jax.experimental.pallas import tpu_sc as plsc

N = 41472
E = 829440
H = 96
EMB = 16
DE = 16

_GW = 32
_GCHUNK = (2 * E) // _GW
_GSUB = 6480
_GNSUB = _GCHUNK // _GSUB


def _gather_cls(nodes, eflat):
    mesh = plsc.VectorSubcoreMesh(core_axis_name="c", subcore_axis_name="s")

    @functools.partial(
        pl.kernel,
        out_type=jax.ShapeDtypeStruct((2 * E,), jnp.int32),
        mesh=mesh,
        scratch_types=[
            pltpu.VMEM_SHARED((N,), jnp.int32),
            pltpu.VMEM((_GSUB,), jnp.int32),
            pltpu.VMEM((_GSUB,), jnp.int32),
            pltpu.VMEM((_GSUB,), jnp.int32),
            pltpu.VMEM((_GSUB,), jnp.int32),
            pltpu.SemaphoreType.DMA((2,)),
            pltpu.SemaphoreType.DMA,
        ],
    )
    def k(nodes_hbm, eflat_hbm, out_hbm, tbl, ibuf0, ibuf1, obuf0, obuf1,
          isem, gsem):
        ibufs = (ibuf0, ibuf1)
        obufs = (obuf0, obuf1)
        cid = lax.axis_index("c")
        sid = lax.axis_index("s")
        wid = sid * 2 + cid
        base = wid * _GCHUNK

        @pl.when(sid == 0)
        def _():
            pltpu.sync_copy(nodes_hbm, tbl)
        plsc.subcore_barrier()

        pltpu.async_copy(eflat_hbm.at[pl.ds(base, _GSUB)], ibufs[0],
                         isem.at[0])
        for sc in range(_GNSUB):
            b = sc % 2
            off = base + sc * _GSUB
            if sc + 1 < _GNSUB:
                pltpu.async_copy(
                    eflat_hbm.at[pl.ds(off + _GSUB, _GSUB)],
                    ibufs[1 - b], isem.at[1 - b])
            pltpu.make_async_copy(eflat_hbm.at[pl.ds(off, _GSUB)],
                                  ibufs[b], isem.at[b]).wait()
            pltpu.async_copy(tbl.at[ibufs[b]], obufs[b], gsem).wait()
            pltpu.sync_copy(obufs[b], out_hbm.at[pl.ds(off, _GSUB)])

    return k(nodes, eflat)


_SG = 24
_SCHUNK_ROWS = 9
_SCHUNK = _SCHUNK_ROWS * 128
_SUPER_ROWS = 16 * _SCHUNK_ROWS
_NSUPER = (E // 128) // _SUPER_ROWS
_NTHIRD = _NSUPER // 3
_ETHIRD = _NTHIRD * _SUPER_ROWS * 128
_SZROWS = 162
_NPT = N // 16


def _scatter_agg(dst2d, msgs2, super0):
    mesh = plsc.VectorSubcoreMesh(core_axis_name="c", subcore_axis_name="s")

    @functools.partial(
        pl.kernel,
        out_type=jax.ShapeDtypeStruct((N, 128), jnp.float32),
        mesh=mesh,
        scratch_types=[
            pltpu.VMEM_SHARED((N, _SG), jnp.float32),
            pltpu.VMEM((_SZROWS, _SG), jnp.float32),
            pltpu.VMEM((2, _SCHUNK, _SG), jnp.float32),
            pltpu.VMEM((2, _SCHUNK_ROWS, 128), jnp.int32),
            pltpu.SemaphoreType.DMA((2,)),
            pltpu.SemaphoreType.DMA((2,)),
            pltpu.SemaphoreType.DMA,
        ],
        compiler_params=pltpu.CompilerParams(use_tc_tiling_on_sc=False),
    )
    def k(dst2d_hbm, msgs2_hbm, out_hbm, acc, zbuf, dbuf, ibuf, dsem, isem,
          ssem):
        cid = lax.axis_index("c")
        sid = lax.axis_index("s")

        zeros16 = jnp.zeros((16,), jnp.float32)

        def zrow(i, carry):
            zbuf[i, pl.ds(0, 16)] = zeros16
            zbuf[i, pl.ds(8, 16)] = zeros16
            return carry

        lax.fori_loop(0, _SZROWS, zrow, 0)

        for p in range(2):
            grp = cid * 2 + p
            col0 = grp * _SG
            for t in range(_NPT // _SZROWS):
                pltpu.sync_copy(
                    zbuf, acc.at[pl.ds(sid * _NPT + t * _SZROWS, _SZROWS)])
            plsc.subcore_barrier()

            def start_in(t, b):
                lrow0 = t * _SUPER_ROWS + sid * _SCHUNK_ROWS
                grow0 = super0 * _SUPER_ROWS + lrow0
                pltpu.async_copy(
                    msgs2_hbm.at[pl.ds(lrow0 * 128, _SCHUNK),
                                 pl.ds(col0, _SG)],
                    dbuf.at[b], dsem.at[b])
                pltpu.async_copy(dst2d_hbm.at[pl.ds(grow0, _SCHUNK_ROWS)],
                                 ibuf.at[b], isem.at[b])

            def wait_in(t, b):
                lrow0 = t * _SUPER_ROWS + sid * _SCHUNK_ROWS
                grow0 = super0 * _SUPER_ROWS + lrow0
                pltpu.make_async_copy(
                    msgs2_hbm.at[pl.ds(lrow0 * 128, _SCHUNK),
                                 pl.ds(col0, _SG)],
                    dbuf.at[b], dsem.at[b]).wait()
                pltpu.make_async_copy(
                    dst2d_hbm.at[pl.ds(grow0, _SCHUNK_ROWS)],
                    ibuf.at[b], isem.at[b]).wait()

            start_in(0, 0)

            def chunk2(t2, carry):
                for b in range(2):
                    t = t2 * 2 + b

                    @pl.when(t < _NTHIRD)
                    def _():
                        @pl.when(t + 1 < _NTHIRD)
                        def _():
                            start_in(t + 1, 1 - b)
                        wait_in(t, b)
                        descs = []
                        for j in range(_SCHUNK_ROWS):
                            descs.append(pltpu.async_copy(
                                dbuf.at[b, pl.ds(j * 128, 128)],
                                acc.at[ibuf.at[b, j]], ssem, add=True))
                        for d in descs:
                            d.wait()
                return carry

            lax.fori_loop(0, (_NTHIRD + 1) // 2, chunk2, 0)
            plsc.subcore_barrier()

            pltpu.sync_copy(acc.at[pl.ds(sid * _NPT, _NPT)],
                            out_hbm.at[pl.ds(sid * _NPT, _NPT),
                                       pl.ds(col0, _SG)])

    return k(dst2d, msgs2)


_EB = 5120
_ENB = E // _EB


def _edge_body(cs_ref, cd_ref, eft_ref, emb_ref, mW0_ref, b0_ref, mW1_ref,
               b1_ref, mW2p_ref, b2p_ref, out_ref):
    f32 = jnp.float32
    dims11 = (((1,), (1,)), ((), ()))
    dims00 = (((0,), (0,)), ((), ()))
    dims01 = (((0,), (1,)), ((), ()))
    cs = cs_ref[0]
    cd = cd_ref[0]
    iota10c = lax.broadcasted_iota(jnp.int32, (10, 1), 0)
    oh_st = (cs == iota10c).astype(f32)
    oh_dt = (cd == iota10c).astype(f32)
    emb = emb_ref[...]
    TA = lax.dot_general(emb, mW0_ref[:, 0:EMB], dims11,
                         preferred_element_type=f32)
    TB = lax.dot_general(emb, mW0_ref[:, EMB:2 * EMB], dims11,
                         preferred_element_type=f32)
    h0 = (lax.dot_general(oh_st, TA, dims00, preferred_element_type=f32)
          + lax.dot_general(oh_dt, TB, dims00, preferred_element_type=f32)
          + lax.dot_general(eft_ref[...], mW0_ref[:, 2 * EMB:], dims01,
                            preferred_element_type=f32)
          + b0_ref[...])
    bf16 = jnp.bfloat16
    h1 = jnp.maximum(h0, 0.0).astype(bf16)
    h2 = jnp.maximum(
        lax.dot_general(h1, mW1_ref[...].astype(bf16), dims11,
                        preferred_element_type=f32)
        + b1_ref[...], 0.0).astype(bf16)
    out_ref[...] = lax.dot_general(h2, mW2p_ref[...].astype(bf16), dims11,
                                   preferred_element_type=f32) + b2p_ref[...]


def _edge_mlp(cs3, cd3, eft, emb_s, mW0, b0r, mW1, b1r, mW2p, b2pr):
    nb = cs3.shape[0]
    ne = nb * _EB
    full = lambda shape: pl.BlockSpec(shape, lambda i, _s=shape: tuple(0 for _ in _s))
    return pl.pallas_call(
        _edge_body,
        grid=(nb,),
        in_specs=[
            pl.BlockSpec((1, 1, _EB), lambda i: (i, 0, 0)),
            pl.BlockSpec((1, 1, _EB), lambda i: (i, 0, 0)),
            pl.BlockSpec((DE, _EB), lambda i: (0, i)),
            full((10, EMB)),
            full((H, 2 * EMB + DE)),
            full((1, H)),
            full((H, H)),
            full((1, H)),
            full((128, H)),
            full((1, 128)),
        ],
        out_specs=pl.BlockSpec((_EB, 128), lambda i: (i, 0)),
        out_shape=jax.ShapeDtypeStruct((ne, 128), jnp.float32),
        compiler_params=pltpu.CompilerParams(
            dimension_semantics=("arbitrary",)),
    )(cs3, cd3, eft, emb_s, mW0, b0r, mW1, b1r, mW2p, b2pr)


_RB = 2304
_RNB = N // _RB


def _post_body(agg0_ref, agg1_ref, agg2_ref, pzt_ref, sht_ref, sct_ref,
               pW0_ref, pb0_ref, pW1_ref,
               pb1_ref, pW2_ref, pb2_ref, Wi_ref, Wf_ref, Wg_ref, Wo_ref,
               Ui_ref, Uf_ref, Ug_ref, Uo_ref, bi_ref, bf_ref, bg_ref,
               bo_ref, oW_ref, ob_ref, h_ref, c_ref, o_ref):
    f32 = jnp.float32
    dims11 = (((1,), (1,)), ((), ()))
    dims10 = (((1,), (0,)), ((), ()))

    def dott(w, x):
        return lax.dot_general(w, x, dims10, preferred_element_type=f32)

    agg96 = agg0_ref[:, 0:H] + agg1_ref[:, 0:H] + agg2_ref[:, 0:H]
    g0 = (lax.dot_general(pW0_ref[:, 0:H], agg96, dims11,
                          preferred_element_type=f32)
          + dott(pW0_ref[:, H:H + EMB], pzt_ref[...]) + pb0_ref[...])
    h = jnp.maximum(g0, 0.0)
    h = jnp.maximum(dott(pW1_ref[...], h) + pb1_ref[...], 0.0)
    hp = dott(pW2_ref[...], h) + pb2_ref[...]
    sh = sht_ref[...]
    ii = jax.nn.sigmoid(dott(Wi_ref[...], hp) + dott(Ui_ref[...], sh) + bi_ref[...])
    ff = jax.nn.sigmoid(dott(Wf_ref[...], hp) + dott(Uf_ref[...], sh) + bf_ref[...])
    gg = jnp.tanh(dott(Wg_ref[...], hp) + dott(Ug_ref[...], sh) + bg_ref[...])
    oo = jax.nn.sigmoid(dott(Wo_ref[...], hp) + dott(Uo_ref[...], sh) + bo_ref[...])
    cn = ff * sct_ref[...] + ii * gg
    hn = oo * jnp.tanh(cn)
    h_ref[...] = hn
    c_ref[...] = cn
    o_ref[...] = dott(oW_ref[...], hn) + ob_ref[...]


def _post(aggs, puzzlet, sht, sct, pW0, pb0c, pW1, pb1c, pW2, pb2c, Ws, Us,
          bs, oW, obc):
    full = lambda shape: pl.BlockSpec(shape, lambda i, _s=shape: tuple(0 for _ in _s))
    return pl.pallas_call(
        _post_body,
        grid=(_RNB,),
        in_specs=[
            pl.BlockSpec((_RB, 128), lambda i: (i, 0)),
            pl.BlockSpec((_RB, 128), lambda i: (i, 0)),
            pl.BlockSpec((_RB, 128), lambda i: (i, 0)),
            pl.BlockSpec((EMB, _RB), lambda i: (0, i)),
            pl.BlockSpec((H, _RB), lambda i: (0, i)),
            pl.BlockSpec((H, _RB), lambda i: (0, i)),
            full((H, H + EMB)),
            full((H, 1)),
            full((H, H)),
            full((H, 1)),
            full((H, H)),
            full((H, 1)),
            *[full((H, H)) for _ in range(8)],
            *[full((H, 1)) for _ in range(4)],
            full((10, H)),
            full((10, 1)),
        ],
        out_specs=[
            pl.BlockSpec((H, _RB), lambda i: (0, i)),
            pl.BlockSpec((H, _RB), lambda i: (0, i)),
            pl.BlockSpec((10, _RB), lambda i: (0, i)),
        ],
        out_shape=[
            jax.ShapeDtypeStruct((H, N), jnp.float32),
            jax.ShapeDtypeStruct((H, N), jnp.float32),
            jax.ShapeDtypeStruct((10, N), jnp.float32),
        ],
        compiler_params=pltpu.CompilerParams(
            dimension_semantics=("arbitrary",)),
    )(*aggs, puzzlet, sht, sct, pW0, pb0c, pW1, pb1c, pW2, pb2c, *Ws, *Us,
      *bs, oW, obc)


def kernel(puzzle, nodes, edges, edge_features, state_h, state_c, first, emb,
           mW0, mb0, mW1, mb1, mW2, mb2, pW0, pb0, pW1, pb1, pW2, pb2, W_ih,
           W_hh, b_ih, b_hh, oW, ob):
    f32 = jnp.float32
    nodes = nodes.astype(jnp.int32)
    eflat = edges.astype(jnp.int32).reshape(2 * E)
    dst2d = edges[1].astype(jnp.int32).reshape(E // 128, 128)

    cls = _gather_cls(nodes, eflat)
    cs3 = cls[:E].reshape(_ENB, 1, _EB)
    cd3 = cls[E:].reshape(_ENB, 1, _EB)
    eft = edge_features.T

    emb_s = emb * jnp.asarray(first, f32)
    mW2p = jnp.concatenate([mW2, jnp.zeros((128 - H, H), f32)], axis=0)
    b2p = jnp.concatenate([mb2, jnp.zeros((128 - H,), f32)]).reshape(1, 128)

    nb3 = _ETHIRD // _EB
    aggs = []
    for k in range(3):
        e0, e1 = k * _ETHIRD, (k + 1) * _ETHIRD
        msgs_k = _edge_mlp(
            cs3[k * nb3:(k + 1) * nb3], cd3[k * nb3:(k + 1) * nb3],
            eft[:, e0:e1], emb_s, mW0, mb0.reshape(1, H), mW1,
            mb1.reshape(1, H), mW2p, b2p)
        aggs.append(_scatter_agg(dst2d, msgs_k, k * _NTHIRD))

    b = b_ih + b_hh
    Ws = [W_ih[i * H:(i + 1) * H] for i in range(4)]
    Us = [W_hh[i * H:(i + 1) * H] for i in range(4)]
    bs = [b[i * H:(i + 1) * H].reshape(H, 1) for i in range(4)]
    ht, ct, outt = _post(
        aggs, puzzle.T, state_h.T, state_c.T, pW0, pb0.reshape(H, 1), pW1,
        pb1.reshape(H, 1), pW2, pb2.reshape(H, 1), Ws, Us, bs, oW,
        ob.reshape(10, 1))
    return (ht.T, ct.T, outt.T.reshape(-1, 81, 10))


# speedup vs baseline: 14.1112x; 1.1165x over previous
import functools

import jax
import jax.numpy as jnp
from jax import lax
from jax.experimental import pallas as pl
from jax.experimental.pallas import tpu as pltpu
from jax.experimental.pallas import tpu_sc as plsc

N = 41472
E = 829440
H = 96
EMB = 16
DE = 16

_GW = 32
_GCHUNK = (2 * E) // _GW
_GSUB = 6480
_GNSUB = _GCHUNK // _GSUB


def _gather_cls(nodes, eflat):
    mesh = plsc.VectorSubcoreMesh(core_axis_name="c", subcore_axis_name="s")

    @functools.partial(
        pl.kernel,
        out_type=jax.ShapeDtypeStruct((2 * E,), jnp.int32),
        mesh=mesh,
        scratch_types=[
            pltpu.VMEM_SHARED((N,), jnp.int32),
            pltpu.VMEM((_GSUB,), jnp.int32),
            pltpu.VMEM((_GSUB,), jnp.int32),
            pltpu.VMEM((_GSUB,), jnp.int32),
            pltpu.VMEM((_GSUB,), jnp.int32),
            pltpu.SemaphoreType.DMA((2,)),
            pltpu.SemaphoreType.DMA,
        ],
    )
    def k(nodes_hbm, eflat_hbm, out_hbm, tbl, ibuf0, ibuf1, obuf0, obuf1,
          isem, gsem):
        ibufs = (ibuf0, ibuf1)
        obufs = (obuf0, obuf1)
        cid = lax.axis_index("c")
        sid = lax.axis_index("s")
        wid = sid * 2 + cid
        base = wid * _GCHUNK

        @pl.when(sid == 0)
        def _():
            pltpu.sync_copy(nodes_hbm, tbl)
        plsc.subcore_barrier()

        pltpu.async_copy(eflat_hbm.at[pl.ds(base, _GSUB)], ibufs[0],
                         isem.at[0])
        for sc in range(_GNSUB):
            b = sc % 2
            off = base + sc * _GSUB
            if sc + 1 < _GNSUB:
                pltpu.async_copy(
                    eflat_hbm.at[pl.ds(off + _GSUB, _GSUB)],
                    ibufs[1 - b], isem.at[1 - b])
            pltpu.make_async_copy(eflat_hbm.at[pl.ds(off, _GSUB)],
                                  ibufs[b], isem.at[b]).wait()
            pltpu.async_copy(tbl.at[ibufs[b]], obufs[b], gsem).wait()
            pltpu.sync_copy(obufs[b], out_hbm.at[pl.ds(off, _GSUB)])

    return k(nodes, eflat)


_SG = 24
_SCHUNK_ROWS = 9
_SCHUNK = _SCHUNK_ROWS * 128
_SUPER_ROWS = 16 * _SCHUNK_ROWS
_NSUPER = (E // 128) // _SUPER_ROWS
_NTHIRD = _NSUPER // 3
_ETHIRD = _NTHIRD * _SUPER_ROWS * 128
_SZROWS = 162
_NPT = N // 16


def _scatter_agg(dst2d, msgs2, super0):
    mesh = plsc.VectorSubcoreMesh(core_axis_name="c", subcore_axis_name="s")

    @functools.partial(
        pl.kernel,
        out_type=jax.ShapeDtypeStruct((N, 128), jnp.float32),
        mesh=mesh,
        scratch_types=[
            pltpu.VMEM_SHARED((N, _SG), jnp.float32),
            pltpu.VMEM((_SZROWS, _SG), jnp.float32),
            pltpu.VMEM((2, _SCHUNK, _SG), jnp.float32),
            pltpu.VMEM((2, _SCHUNK_ROWS, 128), jnp.int32),
            pltpu.SemaphoreType.DMA((2,)),
            pltpu.SemaphoreType.DMA((2,)),
            pltpu.SemaphoreType.DMA,
        ],
        compiler_params=pltpu.CompilerParams(use_tc_tiling_on_sc=False),
    )
    def k(dst2d_hbm, msgs2_hbm, out_hbm, acc, zbuf, dbuf, ibuf, dsem, isem,
          ssem):
        cid = lax.axis_index("c")
        sid = lax.axis_index("s")

        zeros16 = jnp.zeros((16,), jnp.float32)

        def zrow(i, carry):
            zbuf[i, pl.ds(0, 16)] = zeros16
            zbuf[i, pl.ds(8, 16)] = zeros16
            return carry

        lax.fori_loop(0, _SZROWS, zrow, 0)

        for p in range(2):
            grp = cid * 2 + p
            col0 = grp * _SG
            for t in range(_NPT // _SZROWS):
                pltpu.sync_copy(
                    zbuf, acc.at[pl.ds(sid * _NPT + t * _SZROWS, _SZROWS)])
            plsc.subcore_barrier()

            def start_in(t, b):
                lrow0 = t * _SUPER_ROWS + sid * _SCHUNK_ROWS
                grow0 = super0 * _SUPER_ROWS + lrow0
                pltpu.async_copy(
                    msgs2_hbm.at[pl.ds(lrow0 * 128, _SCHUNK),
                                 pl.ds(col0, _SG)],
                    dbuf.at[b], dsem.at[b])
                pltpu.async_copy(dst2d_hbm.at[pl.ds(grow0, _SCHUNK_ROWS)],
                                 ibuf.at[b], isem.at[b])

            def wait_in(t, b):
                lrow0 = t * _SUPER_ROWS + sid * _SCHUNK_ROWS
                grow0 = super0 * _SUPER_ROWS + lrow0
                pltpu.make_async_copy(
                    msgs2_hbm.at[pl.ds(lrow0 * 128, _SCHUNK),
                                 pl.ds(col0, _SG)],
                    dbuf.at[b], dsem.at[b]).wait()
                pltpu.make_async_copy(
                    dst2d_hbm.at[pl.ds(grow0, _SCHUNK_ROWS)],
                    ibuf.at[b], isem.at[b]).wait()

            start_in(0, 0)

            def chunk2(t2, carry):
                for b in range(2):
                    t = t2 * 2 + b

                    @pl.when(t < _NTHIRD)
                    def _():
                        @pl.when(t + 1 < _NTHIRD)
                        def _():
                            start_in(t + 1, 1 - b)
                        wait_in(t, b)
                        descs = []
                        for j in range(_SCHUNK_ROWS):
                            descs.append(pltpu.async_copy(
                                dbuf.at[b, pl.ds(j * 128, 128)],
                                acc.at[ibuf.at[b, j]], ssem, add=True))
                        for d in descs:
                            d.wait()
                return carry

            lax.fori_loop(0, (_NTHIRD + 1) // 2, chunk2, 0)
            plsc.subcore_barrier()

            pltpu.sync_copy(acc.at[pl.ds(sid * _NPT, _NPT)],
                            out_hbm.at[pl.ds(sid * _NPT, _NPT),
                                       pl.ds(col0, _SG)])

    return k(dst2d, msgs2)


_EB = 5120
_ENB = E // _EB


def _edge_body(cs_ref, cd_ref, eft_ref, emb_ref, mW0_ref, b0_ref, mW1_ref,
               b1_ref, mW2p_ref, b2p_ref, out_ref):
    f32 = jnp.float32
    bf16 = jnp.bfloat16
    dims11 = (((1,), (1,)), ((), ()))
    dims00 = (((0,), (0,)), ((), ()))
    dims01 = (((0,), (1,)), ((), ()))
    cs = cs_ref[0]
    cd = cd_ref[0]
    iota10c = lax.broadcasted_iota(jnp.int32, (10, 1), 0)
    oh_st = (cs == iota10c).astype(bf16)
    oh_dt = (cd == iota10c).astype(bf16)
    ohcat = jnp.concatenate([oh_st, oh_dt], axis=0)
    emb = emb_ref[...]
    TA = lax.dot_general(emb, mW0_ref[:, 0:EMB], dims11,
                         preferred_element_type=f32)
    TB = lax.dot_general(emb, mW0_ref[:, EMB:2 * EMB], dims11,
                         preferred_element_type=f32)
    TAB = jnp.concatenate([TA, TB], axis=0)
    h0 = (lax.dot_general(ohcat, TAB.astype(bf16), dims00,
                          preferred_element_type=f32)
          + lax.dot_general(eft_ref[...].astype(bf16),
                            mW0_ref[:, 2 * EMB:].astype(bf16), dims01,
                            preferred_element_type=f32)
          + b0_ref[...])
    h1 = jnp.maximum(h0, 0.0).astype(bf16)
    h2 = jnp.maximum(
        lax.dot_general(h1, mW1_ref[...].astype(bf16), dims11,
                        preferred_element_type=f32)
        + b1_ref[...], 0.0).astype(bf16)
    out_ref[...] = lax.dot_general(h2, mW2p_ref[...].astype(bf16), dims11,
                                   preferred_element_type=f32) + b2p_ref[...]


def _edge_mlp(cs3, cd3, eft, emb_s, mW0, b0r, mW1, b1r, mW2p, b2pr):
    nb = cs3.shape[0]
    ne = nb * _EB
    full = lambda shape: pl.BlockSpec(shape, lambda i, _s=shape: tuple(0 for _ in _s))
    return pl.pallas_call(
        _edge_body,
        grid=(nb,),
        in_specs=[
            pl.BlockSpec((1, 1, _EB), lambda i: (i, 0, 0)),
            pl.BlockSpec((1, 1, _EB), lambda i: (i, 0, 0)),
            pl.BlockSpec((DE, _EB), lambda i: (0, i)),
            full((10, EMB)),
            full((H, 2 * EMB + DE)),
            full((1, H)),
            full((H, H)),
            full((1, H)),
            full((128, H)),
            full((1, 128)),
        ],
        out_specs=pl.BlockSpec((_EB, 128), lambda i: (i, 0)),
        out_shape=jax.ShapeDtypeStruct((ne, 128), jnp.float32),
        compiler_params=pltpu.CompilerParams(
            dimension_semantics=("arbitrary",)),
    )(cs3, cd3, eft, emb_s, mW0, b0r, mW1, b1r, mW2p, b2pr)


_RB = 2304
_RNB = N // _RB


def _post_body(agg0_ref, agg1_ref, agg2_ref, pzt_ref, sht_ref, sct_ref,
               pW0_ref, pb0_ref, pW1_ref,
               pb1_ref, pW2_ref, pb2_ref, Wi_ref, Wf_ref, Wg_ref, Wo_ref,
               Ui_ref, Uf_ref, Ug_ref, Uo_ref, bi_ref, bf_ref, bg_ref,
               bo_ref, oW_ref, ob_ref, h_ref, c_ref, o_ref):
    f32 = jnp.float32
    dims11 = (((1,), (1,)), ((), ()))
    dims10 = (((1,), (0,)), ((), ()))

    def dott(w, x):
        return lax.dot_general(w, x, dims10, preferred_element_type=f32)

    agg96 = agg0_ref[:, 0:H] + agg1_ref[:, 0:H] + agg2_ref[:, 0:H]
    g0 = (lax.dot_general(pW0_ref[:, 0:H], agg96, dims11,
                          preferred_element_type=f32)
          + dott(pW0_ref[:, H:H + EMB], pzt_ref[...]) + pb0_ref[...])
    h = jnp.maximum(g0, 0.0)
    h = jnp.maximum(dott(pW1_ref[...], h) + pb1_ref[...], 0.0)
    hp = dott(pW2_ref[...], h) + pb2_ref[...]
    sh = sht_ref[...]
    ii = jax.nn.sigmoid(dott(Wi_ref[...], hp) + dott(Ui_ref[...], sh) + bi_ref[...])
    ff = jax.nn.sigmoid(dott(Wf_ref[...], hp) + dott(Uf_ref[...], sh) + bf_ref[...])
    gg = jnp.tanh(dott(Wg_ref[...], hp) + dott(Ug_ref[...], sh) + bg_ref[...])
    oo = jax.nn.sigmoid(dott(Wo_ref[...], hp) + dott(Uo_ref[...], sh) + bo_ref[...])
    cn = ff * sct_ref[...] + ii * gg
    hn = oo * jnp.tanh(cn)
    h_ref[...] = hn
    c_ref[...] = cn
    o_ref[...] = dott(oW_ref[...], hn) + ob_ref[...]


def _post(aggs, puzzlet, sht, sct, pW0, pb0c, pW1, pb1c, pW2, pb2c, Ws, Us,
          bs, oW, obc):
    full = lambda shape: pl.BlockSpec(shape, lambda i, _s=shape: tuple(0 for _ in _s))
    return pl.pallas_call(
        _post_body,
        grid=(_RNB,),
        in_specs=[
            pl.BlockSpec((_RB, 128), lambda i: (i, 0)),
            pl.BlockSpec((_RB, 128), lambda i: (i, 0)),
            pl.BlockSpec((_RB, 128), lambda i: (i, 0)),
            pl.BlockSpec((EMB, _RB), lambda i: (0, i)),
            pl.BlockSpec((H, _RB), lambda i: (0, i)),
            pl.BlockSpec((H, _RB), lambda i: (0, i)),
            full((H, H + EMB)),
            full((H, 1)),
            full((H, H)),
            full((H, 1)),
            full((H, H)),
            full((H, 1)),
            *[full((H, H)) for _ in range(8)],
            *[full((H, 1)) for _ in range(4)],
            full((10, H)),
            full((10, 1)),
        ],
        out_specs=[
            pl.BlockSpec((H, _RB), lambda i: (0, i)),
            pl.BlockSpec((H, _RB), lambda i: (0, i)),
            pl.BlockSpec((10, _RB), lambda i: (0, i)),
        ],
        out_shape=[
            jax.ShapeDtypeStruct((H, N), jnp.float32),
            jax.ShapeDtypeStruct((H, N), jnp.float32),
            jax.ShapeDtypeStruct((10, N), jnp.float32),
        ],
        compiler_params=pltpu.CompilerParams(
            dimension_semantics=("arbitrary",)),
    )(*aggs, puzzlet, sht, sct, pW0, pb0c, pW1, pb1c, pW2, pb2c, *Ws, *Us,
      *bs, oW, obc)


def kernel(puzzle, nodes, edges, edge_features, state_h, state_c, first, emb,
           mW0, mb0, mW1, mb1, mW2, mb2, pW0, pb0, pW1, pb1, pW2, pb2, W_ih,
           W_hh, b_ih, b_hh, oW, ob):
    f32 = jnp.float32
    nodes = nodes.astype(jnp.int32)
    eflat = edges.astype(jnp.int32).reshape(2 * E)
    dst2d = edges[1].astype(jnp.int32).reshape(E // 128, 128)

    cls = _gather_cls(nodes, eflat)
    cs3 = cls[:E].reshape(_ENB, 1, _EB)
    cd3 = cls[E:].reshape(_ENB, 1, _EB)
    eft = edge_features.T

    emb_s = emb * jnp.asarray(first, f32)
    mW2p = jnp.concatenate([mW2, jnp.zeros((128 - H, H), f32)], axis=0)
    b2p = jnp.concatenate([mb2, jnp.zeros((128 - H,), f32)]).reshape(1, 128)

    nb3 = _ETHIRD // _EB
    aggs = []
    for k in range(3):
        e0, e1 = k * _ETHIRD, (k + 1) * _ETHIRD
        msgs_k = _edge_mlp(
            cs3[k * nb3:(k + 1) * nb3], cd3[k * nb3:(k + 1) * nb3],
            eft[:, e0:e1], emb_s, mW0, mb0.reshape(1, H), mW1,
            mb1.reshape(1, H), mW2p, b2p)
        aggs.append(_scatter_agg(dst2d, msgs_k, k * _NTHIRD))

    b = b_ih + b_hh
    Ws = [W_ih[i * H:(i + 1) * H] for i in range(4)]
    Us = [W_hh[i * H:(i + 1) * H] for i in range(4)]
    bs = [b[i * H:(i + 1) * H].reshape(H, 1) for i in range(4)]
    ht, ct, outt = _post(
        aggs, puzzle.T, state_h.T, state_c.T, pW0, pb0.reshape(H, 1), pW1,
        pb1.reshape(H, 1), pW2, pb2.reshape(H, 1), Ws, Us, bs, oW,
        ob.reshape(10, 1))
    return (ht.T, ct.T, outt.T.reshape(-1, 81, 10))
